# Initial kernel scaffold; baseline (speedup 1.0000x reference)
import jax
import jax.numpy as jnp
from jax.experimental import pallas as pl


def kernel(h, edge_index, e, snorm_n, snorm_e, W_h, b_h, W_pre, b_pre, W_post, b_post):
    raise NotImplementedError("write your pallas kernel here")


# speedup vs baseline: 1.0100x; 1.0100x over previous
import functools

import jax
import jax.numpy as jnp
from jax.experimental import pallas as pl

N = 10000
E = 320000
D = 128
L = 4
AVG_D_LOG = 3.4965

EB = 2560


def _edge_mlp_body(hs_ref, hd_ref, wa_ref, wb_ref, b_ref, o_ref):
    acc = jnp.dot(hs_ref[...], wa_ref[...], preferred_element_type=jnp.float32)
    acc = acc + jnp.dot(hd_ref[...], wb_ref[...], preferred_element_type=jnp.float32)
    o_ref[...] = jnp.maximum(acc + b_ref[...], 0.0)


def _edge_mlp(hs, hd, w_pre, b_pre):
    wa = w_pre[:D]
    wb = w_pre[D:]
    b2 = b_pre.reshape(1, D)
    grid = (E // EB,)
    return pl.pallas_call(
        _edge_mlp_body,
        grid=grid,
        in_specs=[
            pl.BlockSpec((EB, D), lambda i: (i, 0)),
            pl.BlockSpec((EB, D), lambda i: (i, 0)),
            pl.BlockSpec((D, D), lambda i: (0, 0)),
            pl.BlockSpec((D, D), lambda i: (0, 0)),
            pl.BlockSpec((1, D), lambda i: (0, 0)),
        ],
        out_specs=pl.BlockSpec((EB, D), lambda i: (i, 0)),
        out_shape=jax.ShapeDtypeStruct((E, D), jnp.float32),
    )(hs, hd, wa, wb, b2)


NB = 2000


def _embed_body(h_ref, w_ref, b_ref, o_ref):
    o_ref[...] = (
        jnp.dot(h_ref[...], w_ref[...], preferred_element_type=jnp.float32)
        + b_ref[...]
    )


def _embed(h, w_h, b_h):
    return pl.pallas_call(
        _embed_body,
        grid=(N // NB,),
        in_specs=[
            pl.BlockSpec((NB, D), lambda i: (i, 0)),
            pl.BlockSpec((D, D), lambda i: (0, 0)),
            pl.BlockSpec((1, D), lambda i: (0, 0)),
        ],
        out_specs=pl.BlockSpec((NB, D), lambda i: (i, 0)),
        out_shape=jax.ShapeDtypeStruct((N, D), jnp.float32),
    )(h, w_h, b_h.reshape(1, D))


def _post_a_body(x_ref, s1_ref, s2_ref, mx_ref, mn_ref, cnt_ref, sn_ref,
                 w_ref, b_ref, y_ref, cs_ref, css_ref):
    cnt = cnt_ref[...]
    d = jnp.maximum(cnt, 1.0)
    inv_d = 1.0 / d
    mean = s1_ref[...] * inv_d
    var = jnp.maximum(s2_ref[...] * inv_d - mean * mean, 0.0)
    std = jnp.sqrt(var + 1e-5)
    has = cnt > 0.0
    mx = jnp.where(has, mx_ref[...], 0.0)
    mn = jnp.where(has, mn_ref[...], 0.0)
    logd = jnp.log(d + 1.0)
    amp = logd * (1.0 / AVG_D_LOG)
    att = AVG_D_LOG / logd
    w = w_ref[...]

    acc = jnp.dot(x_ref[...], w[0:D], preferred_element_type=jnp.float32)
    acc_a = jnp.zeros_like(acc)
    acc_t = jnp.zeros_like(acc)
    stats = (mean, mx, mn, std)
    for k in range(4):
        s = stats[k]
        acc = acc + jnp.dot(s, w[D + k * D:D + (k + 1) * D],
                            preferred_element_type=jnp.float32)
        acc_a = acc_a + jnp.dot(s, w[5 * D + k * D:5 * D + (k + 1) * D],
                                preferred_element_type=jnp.float32)
        acc_t = acc_t + jnp.dot(s, w[9 * D + k * D:9 * D + (k + 1) * D],
                                preferred_element_type=jnp.float32)
    y = (acc + amp * acc_a + att * acc_t + b_ref[...]) * sn_ref[...]
    y_ref[...] = y

    @pl.when(pl.program_id(0) == 0)
    def _init():
        cs_ref[...] = jnp.zeros_like(cs_ref)
        css_ref[...] = jnp.zeros_like(css_ref)

    cs_ref[...] += jnp.sum(y, axis=0, keepdims=True)
    css_ref[...] += jnp.sum(y * y, axis=0, keepdims=True)


def _post_b_body(x_ref, y_ref, cs_ref, css_ref, o_ref):
    mu = cs_ref[...] * (1.0 / N)
    vv = css_ref[...] * (1.0 / N) - mu * mu
    yn = (y_ref[...] - mu) * jax.lax.rsqrt(vv + 1e-5)
    o_ref[...] = x_ref[...] + jnp.maximum(yn, 0.0)


def _post(x, s1, s2, mx, mn, cnt, snorm_n, w_post, b_post):
    grid = (N // NB,)
    nspec = pl.BlockSpec((NB, D), lambda i: (i, 0))
    one_spec = pl.BlockSpec((NB, 1), lambda i: (i, 0))
    col_spec = pl.BlockSpec((1, D), lambda i: (0, 0))
    y, cs, css = pl.pallas_call(
        _post_a_body,
        grid=grid,
        in_specs=[nspec, nspec, nspec, nspec, nspec, one_spec, one_spec,
                  pl.BlockSpec((13 * D, D), lambda i: (0, 0)), col_spec],
        out_specs=[nspec, col_spec, col_spec],
        out_shape=[
            jax.ShapeDtypeStruct((N, D), jnp.float32),
            jax.ShapeDtypeStruct((1, D), jnp.float32),
            jax.ShapeDtypeStruct((1, D), jnp.float32),
        ],
    )(x, s1, s2, mx, mn, cnt.reshape(N, 1), snorm_n, w_post,
      b_post.reshape(1, D))
    return pl.pallas_call(
        _post_b_body,
        grid=grid,
        in_specs=[nspec, nspec, col_spec, col_spec],
        out_specs=nspec,
        out_shape=jax.ShapeDtypeStruct((N, D), jnp.float32),
    )(x, y, cs, css)


def kernel(h, edge_index, e, snorm_n, snorm_e, W_h, b_h, W_pre, b_pre,
           W_post, b_post):
    src = edge_index[0].astype(jnp.int32)
    dst = edge_index[1].astype(jnp.int32)
    x = _embed(h, W_h, b_h)
    ones = jnp.ones((E,), dtype=jnp.float32)
    cnt = jax.ops.segment_sum(ones, dst, num_segments=N)
    for l in range(L):
        hs = x[src]
        hd = x[dst]
        m = _edge_mlp(hs, hd, W_pre[l], b_pre[l])
        s1 = jax.ops.segment_sum(m, dst, num_segments=N)
        s2 = jax.ops.segment_sum(m * m, dst, num_segments=N)
        mx = jax.ops.segment_max(m, dst, num_segments=N)
        mx = jnp.where(cnt[:, None] > 0, mx, 0.0)
        mn = jax.ops.segment_min(m, dst, num_segments=N)
        mn = jnp.where(cnt[:, None] > 0, mn, 0.0)
        x = _post(x, s1, s2, mx, mn, cnt, snorm_n, W_post[l], b_post[l])
    return x


# speedup vs baseline: 2.3022x; 2.2794x over previous
import functools

import jax
import jax.numpy as jnp
from jax import lax
from jax.experimental import pallas as pl
from jax.experimental.pallas import tpu as pltpu
from jax.experimental.pallas import tpu_sc as plsc

N = 10000
E = 320000
D = 128
L = 4
AVG_D_LOG = 3.4965

NW = 32
PT = E // NW
EB = 2560
NB = 2000
GC = 80
CC = 80
CH = 400
K = 16
NPAD = 10240
NG = 8


def _mesh():
    return plsc.VectorSubcoreMesh(core_axis_name="c", subcore_axis_name="s")


def _wid():
    return lax.axis_index("s") * 2 + lax.axis_index("c")


def _cnt_body(dst_hbm, cnt_hbm, di_v, ones_v, z_v, acc_sh):
    sid = lax.axis_index("s")
    cid = lax.axis_index("c")
    wid = sid * 2 + cid

    def fill(i, carry):
        ones_v[pl.ds(i * 16, 16)] = jnp.full((16,), 1.0, jnp.float32)
        return carry

    lax.fori_loop(0, CC // 16, fill, 0)

    def fillz(i, carry):
        z_v[pl.ds(i * 16, 16)] = jnp.zeros((16,), jnp.float32)
        return carry

    lax.fori_loop(0, 2000 // 16, fillz, 0)

    @pl.when(sid == 0)
    def _zero():
        def zc(i, carry):
            pltpu.sync_copy(z_v, acc_sh.at[pl.ds(i * 2000, 2000)])
            return carry

        lax.fori_loop(0, N // 2000, zc, 0)

    plsc.subcore_barrier()

    start = wid * PT

    def chunk(i, carry):
        pltpu.sync_copy(dst_hbm.at[pl.ds(start + i * CC, CC)], di_v)
        pltpu.sync_copy(ones_v, acc_sh.at[di_v], add=True)
        return carry

    lax.fori_loop(0, PT // CC, chunk, 0)
    plsc.subcore_barrier()

    @pl.when(sid == 0)
    def _export():
        pltpu.sync_copy(acc_sh, cnt_hbm.at[cid])


def _cnt_sc(sdst):
    k = functools.partial(
        pl.kernel,
        mesh=_mesh(),
        out_type=jax.ShapeDtypeStruct((2, N), jnp.float32),
        scratch_types=[
            pltpu.VMEM((CC,), jnp.int32),
            pltpu.VMEM((CC,), jnp.float32),
            pltpu.VMEM((2000,), jnp.float32),
            pltpu.VMEM_SHARED((N,), jnp.float32),
        ],
    )(_cnt_body)
    return k(sdst)


def _gather_body(x_hbm, src_hbm, dst_hbm, hs_hbm, hd_hbm,
                 si_v, di_v, hs_v, hd_v, sem1, sem2):
    start = _wid() * PT

    def chunk(i, carry):
        base = start + i * GC
        pltpu.sync_copy(src_hbm.at[pl.ds(base, GC)], si_v)
        pltpu.sync_copy(dst_hbm.at[pl.ds(base, GC)], di_v)
        c1 = pltpu.async_copy(x_hbm.at[si_v], hs_v, sem1)
        c2 = pltpu.async_copy(x_hbm.at[di_v], hd_v, sem2)
        c1.wait()
        c2.wait()
        pltpu.sync_copy(hs_v, hs_hbm.at[pl.ds(base, GC)])
        pltpu.sync_copy(hd_v, hd_hbm.at[pl.ds(base, GC)])
        return carry

    lax.fori_loop(0, PT // GC, chunk, 0)


def _gather_sc(x, ssrc, sdst):
    k = functools.partial(
        pl.kernel,
        mesh=_mesh(),
        out_type=[jax.ShapeDtypeStruct((E, D), jnp.float32),
                  jax.ShapeDtypeStruct((E, D), jnp.float32)],
        scratch_types=[
            pltpu.VMEM((GC,), jnp.int32),
            pltpu.VMEM((GC,), jnp.int32),
            pltpu.VMEM((GC, D), jnp.float32),
            pltpu.VMEM((GC, D), jnp.float32),
            pltpu.SemaphoreType.DMA,
            pltpu.SemaphoreType.DMA,
        ],
    )(_gather_body)
    return k(x, ssrc, sdst)


def _stats_body(m_hbm, dst_hbm, s1_hbm, s2_hbm, mx_hbm, mn_hbm,
                rec_hbm, rid_hbm,
                m_v, d_v, st_s, st_q, st_x, st_n, ids_v, rec_v, rid_v,
                sem1, sem2, sem3, sem4):
    wid = _wid()
    start = wid * PT
    lanes = lax.iota(jnp.int32, 16)
    dump = jnp.full((16,), N, jnp.int32) + lanes

    for t in range(NG):
        rec_v[pl.ds(0 * D + t * 16, 16)] = jnp.zeros((16,), jnp.float32)
        rec_v[pl.ds(1 * D + t * 16, 16)] = jnp.zeros((16,), jnp.float32)
        rec_v[pl.ds(2 * D + t * 16, 16)] = jnp.full((16,), -jnp.inf, jnp.float32)
        rec_v[pl.ds(3 * D + t * 16, 16)] = jnp.full((16,), jnp.inf, jnp.float32)

    def edge_body(j, carry, base):
        (cur, own, p, ids, rid) = carry[:5]
        accs = carry[5]
        d = d_v[pl.ds(j, 16)][0]
        is_new = d != cur
        real = jnp.logical_and(is_new, cur >= 0)
        flush = jnp.logical_and(real, own)
        head = jnp.logical_and(real, jnp.logical_not(own))

        @pl.when(flush)
        def _stage():
            for t in range(NG):
                st_s[pl.ds(p * D + t * 16, 16)] = accs[0][t]
                st_q[pl.ds(p * D + t * 16, 16)] = accs[1][t]
                st_x[pl.ds(p * D + t * 16, 16)] = accs[2][t]
                st_n[pl.ds(p * D + t * 16, 16)] = accs[3][t]

        @pl.when(head)
        def _head_rec():
            for t in range(NG):
                rec_v[pl.ds(0 * D + t * 16, 16)] = accs[0][t]
                rec_v[pl.ds(1 * D + t * 16, 16)] = accs[1][t]
                rec_v[pl.ds(2 * D + t * 16, 16)] = accs[2][t]
                rec_v[pl.ds(3 * D + t * 16, 16)] = accs[3][t]

        rid2 = jnp.where(head, jnp.where(lanes == 0, cur, rid), rid)
        ids2 = jnp.where(flush, jnp.where(lanes == p, cur, ids), ids)
        p2 = jnp.where(flush, p + 1, p)
        do_drain = jnp.logical_and(flush, p2 == K)

        @pl.when(do_drain)
        def _drain():
            ids_v[pl.ds(0, 16)] = ids2
            hs = []
            for r in range(K):
                idr = ids_v[pl.ds(r, 16)][0]
                rsl = pl.ds(r * D, D)
                hs.append(pltpu.async_copy(st_s.at[rsl], s1_hbm.at[idr], sem1))
                hs.append(pltpu.async_copy(st_q.at[rsl], s2_hbm.at[idr], sem2))
                hs.append(pltpu.async_copy(st_x.at[rsl], mx_hbm.at[idr], sem3))
                hs.append(pltpu.async_copy(st_n.at[rsl], mn_hbm.at[idr], sem4))
            for h in hs:
                h.wait()

        p3 = jnp.where(do_drain, 0, p2)
        ids3 = jnp.where(do_drain, dump, ids2)
        own2 = jnp.logical_or(own, real)
        cur2 = jnp.where(is_new, d, cur)

        new_accs = ([], [], [], [])
        for t in range(NG):
            v = m_v[pl.ds(j * D + t * 16, 16)]
            q = v * v
            new_accs[0].append(jnp.where(is_new, v, accs[0][t] + v))
            new_accs[1].append(jnp.where(is_new, q, accs[1][t] + q))
            new_accs[2].append(jnp.where(is_new, v, jnp.maximum(accs[2][t], v)))
            new_accs[3].append(jnp.where(is_new, v, jnp.minimum(accs[3][t], v)))
        return (cur2, own2, p3, ids3, rid2, new_accs)

    def chunk_body(k, carry):
        base = start + k * CH
        pltpu.sync_copy(m_hbm.at[pl.ds(base * D, CH * D)], m_v)
        pltpu.sync_copy(dst_hbm.at[pl.ds(base, CH)], d_v.at[pl.ds(0, CH)])
        return lax.fori_loop(0, CH, lambda j, c: edge_body(j, c, base), carry)

    zero = jnp.zeros((16,), jnp.float32)
    accs0 = ([zero] * NG, [zero] * NG, [zero] * NG, [zero] * NG)
    init = (jnp.int32(-1), jnp.bool_(False), jnp.int32(0), dump, dump, accs0)
    carry = lax.fori_loop(0, PT // CH, chunk_body, init)
    (final_cur, final_own, final_p, final_ids, final_rid, final_accs) = carry

    for t in range(NG):
        rec_v[pl.ds(4 * D + t * 16, 16)] = final_accs[0][t]
        rec_v[pl.ds(5 * D + t * 16, 16)] = final_accs[1][t]
        rec_v[pl.ds(6 * D + t * 16, 16)] = final_accs[2][t]
        rec_v[pl.ds(7 * D + t * 16, 16)] = final_accs[3][t]
    rid_f = jnp.where(jnp.logical_not(final_own),
                      jnp.where(lanes == 0, final_cur, final_rid), final_rid)
    rid_f = jnp.where(lanes == 1, final_cur, rid_f)
    rid_v[...] = rid_f
    pltpu.sync_copy(rec_v, rec_hbm.at[pl.ds(wid * 8 * D, 8 * D)])
    pltpu.sync_copy(rid_v, rid_hbm.at[pl.ds(wid * 16, 16)])

    ids_v[pl.ds(0, 16)] = final_ids
    hs = []
    for r in range(K):
        idr = ids_v[pl.ds(r, 16)][0]
        rsl = pl.ds(r * D, D)
        hs.append(pltpu.async_copy(st_s.at[rsl], s1_hbm.at[idr], sem1))
        hs.append(pltpu.async_copy(st_q.at[rsl], s2_hbm.at[idr], sem2))
        hs.append(pltpu.async_copy(st_x.at[rsl], mx_hbm.at[idr], sem3))
        hs.append(pltpu.async_copy(st_n.at[rsl], mn_hbm.at[idr], sem4))
    for h in hs:
        h.wait()


def _stats_sc(m, sdst):
    k = functools.partial(
        pl.kernel,
        mesh=_mesh(),
        out_type=[jax.ShapeDtypeStruct((NPAD, D), jnp.float32)
                  for _ in range(4)]
        + [jax.ShapeDtypeStruct((NW * 8 * D,), jnp.float32),
           jax.ShapeDtypeStruct((NW * 16,), jnp.int32)],
        scratch_types=[
            pltpu.VMEM((CH * D,), jnp.float32),
            pltpu.VMEM((CH + 16,), jnp.int32),
            pltpu.VMEM((K * D,), jnp.float32),
            pltpu.VMEM((K * D,), jnp.float32),
            pltpu.VMEM((K * D,), jnp.float32),
            pltpu.VMEM((K * D,), jnp.float32),
            pltpu.VMEM((32,), jnp.int32),
            pltpu.VMEM((8 * D,), jnp.float32),
            pltpu.VMEM((16,), jnp.int32),
            pltpu.SemaphoreType.DMA,
            pltpu.SemaphoreType.DMA,
            pltpu.SemaphoreType.DMA,
            pltpu.SemaphoreType.DMA,
        ],
    )(_stats_body)
    return k(m.reshape(E * D), sdst)


NR = 2 * NW
RPT = NPAD // NW


def _merge_body(rec_hbm, rid_hbm, s1i, s2i, mxi, mni,
                s1o, s2o, mxo, mno,
                rec_v, rid_v, mg_s, mg_q, mg_x, mg_n, mid_v, buf_v):
    wid = _wid()
    lanes = lax.iota(jnp.int32, 16)
    pltpu.sync_copy(rec_hbm, rec_v)
    pltpu.sync_copy(rid_hbm, rid_v.at[pl.ds(0, NR * 8)])

    def rec_body(r, carry):
        (cur, q, bank) = carry[:3]
        accs = carry[3]
        trow = (r // 2) * 16 + (r % 2)
        idr = rid_v[pl.ds(trow, 16)][0]
        is_new = idr != cur
        flushq = jnp.logical_and(is_new, cur >= 0)

        @pl.when(flushq)
        def _fl():
            for t in range(NG):
                mg_s[pl.ds(q * D + t * 16, 16)] = accs[0][t]
                mg_q[pl.ds(q * D + t * 16, 16)] = accs[1][t]
                mg_x[pl.ds(q * D + t * 16, 16)] = accs[2][t]
                mg_n[pl.ds(q * D + t * 16, 16)] = accs[3][t]

        bank2 = jnp.where(flushq, jnp.where(lanes == q % 16, cur, bank), bank)
        q2 = jnp.where(flushq, q + 1, q)
        bfull = jnp.logical_and(flushq, q2 % 16 == 0)

        @pl.when(bfull)
        def _bank():
            mid_v[pl.ds(q2 - 16, 16)] = bank2

        bank3 = jnp.where(bfull, jnp.full((16,), N, jnp.int32), bank2)
        cur2 = jnp.where(is_new, idr, cur)
        row = (r // 2) * 8 + (r % 2) * 4
        new_accs = ([], [], [], [])
        for t in range(NG):
            vs = rec_v[pl.ds(row * D + t * 16, 16)]
            vq = rec_v[pl.ds((row + 1) * D + t * 16, 16)]
            vx = rec_v[pl.ds((row + 2) * D + t * 16, 16)]
            vn = rec_v[pl.ds((row + 3) * D + t * 16, 16)]
            new_accs[0].append(jnp.where(is_new, vs, accs[0][t] + vs))
            new_accs[1].append(jnp.where(is_new, vq, accs[1][t] + vq))
            new_accs[2].append(jnp.where(is_new, vx,
                                         jnp.maximum(accs[2][t], vx)))
            new_accs[3].append(jnp.where(is_new, vn,
                                         jnp.minimum(accs[3][t], vn)))
        return (cur2, q2, bank3, new_accs)

    zero = jnp.zeros((16,), jnp.float32)
    accs0 = ([zero] * NG, [zero] * NG, [zero] * NG, [zero] * NG)
    init = (jnp.int32(-1), jnp.int32(0), jnp.full((16,), N, jnp.int32), accs0)
    (cur_f, q_f, bank_f, accs_f) = lax.fori_loop(0, NR, rec_body, init)

    for t in range(NG):
        mg_s[pl.ds(q_f * D + t * 16, 16)] = accs_f[0][t]
        mg_q[pl.ds(q_f * D + t * 16, 16)] = accs_f[1][t]
        mg_x[pl.ds(q_f * D + t * 16, 16)] = accs_f[2][t]
        mg_n[pl.ds(q_f * D + t * 16, 16)] = accs_f[3][t]
    bank_l = jnp.where(lanes == q_f % 16, cur_f, bank_f)
    mid_v[pl.ds((q_f // 16) * 16, 16)] = bank_l
    q_n = q_f + 1

    lo = wid * RPT
    for (s_in, s_out, mg) in ((s1i, s1o, mg_s), (s2i, s2o, mg_q),
                              (mxi, mxo, mg_x), (mni, mno, mg_n)):
        pltpu.sync_copy(s_in.at[pl.ds(lo * D, RPT * D)], buf_v)

        def ov_body(s, carry, mg=mg):
            mid = mid_v[pl.ds(s, 16)][0]
            hit = jnp.logical_and(
                s < q_n,
                jnp.logical_and(mid >= lo, mid < lo + RPT))

            @pl.when(hit)
            def _ov():
                for t in range(NG):
                    buf_v[pl.ds((mid - lo) * D + t * 16, 16)] = (
                        mg[pl.ds(s * D + t * 16, 16)])

            return carry

        lax.fori_loop(0, NR + 1, ov_body, 0)
        pltpu.sync_copy(buf_v, s_out.at[pl.ds(lo * D, RPT * D)])


def _merge_sc(rec, rid, s1, s2, mx, mn):
    k = functools.partial(
        pl.kernel,
        mesh=_mesh(),
        out_type=[jax.ShapeDtypeStruct((NPAD * D,), jnp.float32)
                  for _ in range(4)],
        scratch_types=[
            pltpu.VMEM((NR * 4 * D,), jnp.float32),
            pltpu.VMEM((NR * 8 + 16,), jnp.int32),
            pltpu.VMEM(((NR + 16) * D,), jnp.float32),
            pltpu.VMEM(((NR + 16) * D,), jnp.float32),
            pltpu.VMEM(((NR + 16) * D,), jnp.float32),
            pltpu.VMEM(((NR + 16) * D,), jnp.float32),
            pltpu.VMEM((NR + 32,), jnp.int32),
            pltpu.VMEM((RPT * D,), jnp.float32),
        ],
    )(_merge_body)
    out = k(rec, rid, s1.reshape(NPAD * D), s2.reshape(NPAD * D),
            mx.reshape(NPAD * D), mn.reshape(NPAD * D))
    return [o.reshape(NPAD, D) for o in out]


def _edge_mlp_body(hs_ref, hd_ref, wa_ref, wb_ref, b_ref, o_ref):
    acc = jnp.dot(hs_ref[...], wa_ref[...], preferred_element_type=jnp.float32)
    acc = acc + jnp.dot(hd_ref[...], wb_ref[...], preferred_element_type=jnp.float32)
    o_ref[...] = jnp.maximum(acc + b_ref[...], 0.0)


def _edge_mlp(hs, hd, w_pre, b_pre):
    return pl.pallas_call(
        _edge_mlp_body,
        grid=(E // EB,),
        in_specs=[
            pl.BlockSpec((EB, D), lambda i: (i, 0)),
            pl.BlockSpec((EB, D), lambda i: (i, 0)),
            pl.BlockSpec((D, D), lambda i: (0, 0)),
            pl.BlockSpec((D, D), lambda i: (0, 0)),
            pl.BlockSpec((1, D), lambda i: (0, 0)),
        ],
        out_specs=pl.BlockSpec((EB, D), lambda i: (i, 0)),
        out_shape=jax.ShapeDtypeStruct((E, D), jnp.float32),
    )(hs, hd, w_pre[:D], w_pre[D:], b_pre.reshape(1, D))


def _embed_body(h_ref, w_ref, b_ref, o_ref):
    o_ref[...] = (
        jnp.dot(h_ref[...], w_ref[...], preferred_element_type=jnp.float32)
        + b_ref[...]
    )


def _embed(h, w_h, b_h):
    return pl.pallas_call(
        _embed_body,
        grid=(N // NB,),
        in_specs=[
            pl.BlockSpec((NB, D), lambda i: (i, 0)),
            pl.BlockSpec((D, D), lambda i: (0, 0)),
            pl.BlockSpec((1, D), lambda i: (0, 0)),
        ],
        out_specs=pl.BlockSpec((NB, D), lambda i: (i, 0)),
        out_shape=jax.ShapeDtypeStruct((N, D), jnp.float32),
    )(h, w_h, b_h.reshape(1, D))


def _post_a_body(x_ref, s1_ref, s2_ref, mx_ref, mn_ref, c0_ref, c1_ref,
                 sn_ref, w_ref, b_ref, y_ref, cs_ref, css_ref):
    cnt = c0_ref[...] + c1_ref[...]
    d = jnp.maximum(cnt, 1.0)
    inv_d = 1.0 / d
    has = cnt > 0.0
    mean = jnp.where(has, s1_ref[...] * inv_d, 0.0)
    var = jnp.where(has,
                    jnp.maximum(s2_ref[...] * inv_d - mean * mean, 0.0), 0.0)
    std = jnp.sqrt(var + 1e-5)
    mx = jnp.where(has, mx_ref[...], 0.0)
    mn = jnp.where(has, mn_ref[...], 0.0)
    logd = jnp.log(d + 1.0)
    amp = logd * (1.0 / AVG_D_LOG)
    att = AVG_D_LOG / logd
    w = w_ref[...]

    acc = jnp.dot(x_ref[...], w[0:D], preferred_element_type=jnp.float32)
    acc_a = jnp.zeros_like(acc)
    acc_t = jnp.zeros_like(acc)
    stats = (mean, mx, mn, std)
    for k in range(4):
        s = stats[k]
        acc = acc + jnp.dot(s, w[D + k * D:D + (k + 1) * D],
                            preferred_element_type=jnp.float32)
        acc_a = acc_a + jnp.dot(s, w[5 * D + k * D:5 * D + (k + 1) * D],
                                preferred_element_type=jnp.float32)
        acc_t = acc_t + jnp.dot(s, w[9 * D + k * D:9 * D + (k + 1) * D],
                                preferred_element_type=jnp.float32)
    y = (acc + amp * acc_a + att * acc_t + b_ref[...]) * sn_ref[...]
    y_ref[...] = y

    @pl.when(pl.program_id(0) == 0)
    def _init():
        cs_ref[...] = jnp.zeros_like(cs_ref)
        css_ref[...] = jnp.zeros_like(css_ref)

    cs_ref[...] += jnp.sum(y, axis=0, keepdims=True)
    css_ref[...] += jnp.sum(y * y, axis=0, keepdims=True)


def _post_b_body(x_ref, y_ref, cs_ref, css_ref, o_ref):
    mu = cs_ref[...] * (1.0 / N)
    vv = css_ref[...] * (1.0 / N) - mu * mu
    yn = (y_ref[...] - mu) * jax.lax.rsqrt(vv + 1e-5)
    o_ref[...] = x_ref[...] + jnp.maximum(yn, 0.0)


def _post(x, s1, s2, mx, mn, c0, c1, snorm_n, w_post, b_post):
    grid = (N // NB,)
    nspec = pl.BlockSpec((NB, D), lambda i: (i, 0))
    one_spec = pl.BlockSpec((NB, 1), lambda i: (i, 0))
    col_spec = pl.BlockSpec((1, D), lambda i: (0, 0))
    y, cs, css = pl.pallas_call(
        _post_a_body,
        grid=grid,
        in_specs=[nspec, nspec, nspec, nspec, nspec, one_spec, one_spec,
                  one_spec, pl.BlockSpec((13 * D, D), lambda i: (0, 0)),
                  col_spec],
        out_specs=[nspec, col_spec, col_spec],
        out_shape=[
            jax.ShapeDtypeStruct((N, D), jnp.float32),
            jax.ShapeDtypeStruct((1, D), jnp.float32),
            jax.ShapeDtypeStruct((1, D), jnp.float32),
        ],
    )(x, s1, s2, mx, mn, c0, c1, snorm_n, w_post, b_post.reshape(1, D))
    return pl.pallas_call(
        _post_b_body,
        grid=grid,
        in_specs=[nspec, nspec, col_spec, col_spec],
        out_specs=nspec,
        out_shape=jax.ShapeDtypeStruct((N, D), jnp.float32),
    )(x, y, cs, css)


def kernel(h, edge_index, e, snorm_n, snorm_e, W_h, b_h, W_pre, b_pre,
           W_post, b_post):
    src = edge_index[0].astype(jnp.int32)
    dst = edge_index[1].astype(jnp.int32)
    perm = jnp.argsort(dst)
    sdst = dst[perm]
    ssrc = src[perm]
    cnt2 = _cnt_sc(sdst)
    c0 = cnt2[0].reshape(N, 1)
    c1 = cnt2[1].reshape(N, 1)
    x = _embed(h, W_h, b_h)
    for l in range(L):
        hs, hd = _gather_sc(x, ssrc, sdst)
        m = _edge_mlp(hs, hd, W_pre[l], b_pre[l])
        s1p, s2p, mxp, mnp, rec, rid = _stats_sc(m, sdst)
        s1, s2, mx, mn = _merge_sc(rec, rid, s1p, s2p, mxp, mnp)
        x = _post(x, s1, s2, mx, mn, c0, c1, snorm_n, W_post[l], b_post[l])
    return x


# speedup vs baseline: 2.6803x; 1.1642x over previous
import functools

import jax
import jax.numpy as jnp
from jax import lax
from jax.experimental import pallas as pl
from jax.experimental.pallas import tpu as pltpu
from jax.experimental.pallas import tpu_sc as plsc

N = 10000
E = 320000
D = 128
L = 4
AVG_D_LOG = 3.4965

NW = 32
PT = E // NW
EB = 2560
NB = 2000
GC = 80
CC = 80
CH = 400
K = 16
NPAD = 10240
NG = 8


def _mesh():
    return plsc.VectorSubcoreMesh(core_axis_name="c", subcore_axis_name="s")


def _wid():
    return lax.axis_index("s") * 2 + lax.axis_index("c")


def _cnt_body(dst_hbm, cnt_hbm, di_v, ones_v, z_v, acc_sh):
    sid = lax.axis_index("s")
    cid = lax.axis_index("c")
    wid = sid * 2 + cid

    def fill(i, carry):
        ones_v[pl.ds(i * 16, 16)] = jnp.full((16,), 1.0, jnp.float32)
        return carry

    lax.fori_loop(0, CC // 16, fill, 0)

    def fillz(i, carry):
        z_v[pl.ds(i * 16, 16)] = jnp.zeros((16,), jnp.float32)
        return carry

    lax.fori_loop(0, 2000 // 16, fillz, 0)

    @pl.when(sid == 0)
    def _zero():
        def zc(i, carry):
            pltpu.sync_copy(z_v, acc_sh.at[pl.ds(i * 2000, 2000)])
            return carry

        lax.fori_loop(0, N // 2000, zc, 0)

    plsc.subcore_barrier()

    start = wid * PT

    def chunk(i, carry):
        pltpu.sync_copy(dst_hbm.at[pl.ds(start + i * CC, CC)], di_v)
        pltpu.sync_copy(ones_v, acc_sh.at[di_v], add=True)
        return carry

    lax.fori_loop(0, PT // CC, chunk, 0)
    plsc.subcore_barrier()

    @pl.when(sid == 0)
    def _export():
        pltpu.sync_copy(acc_sh, cnt_hbm.at[cid])


def _cnt_sc(sdst):
    k = functools.partial(
        pl.kernel,
        mesh=_mesh(),
        out_type=jax.ShapeDtypeStruct((2, N), jnp.float32),
        scratch_types=[
            pltpu.VMEM((CC,), jnp.int32),
            pltpu.VMEM((CC,), jnp.float32),
            pltpu.VMEM((2000,), jnp.float32),
            pltpu.VMEM_SHARED((N,), jnp.float32),
        ],
    )(_cnt_body)
    return k(sdst)


GB = 400


def _gather_body(x_hbm, src_hbm, dst_hbm, hs_hbm, hd_hbm,
                 si_v, di_v, hs_v, hd_v, sg, sw):
    start = _wid() * PT
    nch = PT // GB

    def chunk(i, carry):
        base = start + i * GB

        @pl.when(i > 0)
        def _w():
            prev = start + (i - 1) * GB
            pltpu.make_async_copy(
                hs_v, hs_hbm.at[pl.ds(prev, GB)], sw).wait()
            pltpu.make_async_copy(
                hd_v, hd_hbm.at[pl.ds(prev, GB)], sw).wait()

        pltpu.sync_copy(src_hbm.at[pl.ds(base, GB)], si_v)
        pltpu.sync_copy(dst_hbm.at[pl.ds(base, GB)], di_v)
        hs = []
        for g in range(GB // 80):
            sl = pl.ds(g * 80, 80)
            hs.append(pltpu.async_copy(
                x_hbm.at[si_v.at[sl]], hs_v.at[sl], sg))
            hs.append(pltpu.async_copy(
                x_hbm.at[di_v.at[sl]], hd_v.at[sl], sg))
        for h in hs:
            h.wait()
        pltpu.async_copy(hs_v, hs_hbm.at[pl.ds(base, GB)], sw)
        pltpu.async_copy(hd_v, hd_hbm.at[pl.ds(base, GB)], sw)
        return carry

    lax.fori_loop(0, nch, chunk, 0)
    last = start + (nch - 1) * GB
    pltpu.make_async_copy(hs_v, hs_hbm.at[pl.ds(last, GB)], sw).wait()
    pltpu.make_async_copy(hd_v, hd_hbm.at[pl.ds(last, GB)], sw).wait()


def _gather_sc(x, ssrc, sdst):
    k = functools.partial(
        pl.kernel,
        mesh=_mesh(),
        out_type=[jax.ShapeDtypeStruct((E, D), jnp.float32),
                  jax.ShapeDtypeStruct((E, D), jnp.float32)],
        scratch_types=[
            pltpu.VMEM((GB,), jnp.int32),
            pltpu.VMEM((GB,), jnp.int32),
            pltpu.VMEM((GB, D), jnp.float32),
            pltpu.VMEM((GB, D), jnp.float32),
            pltpu.SemaphoreType.DMA,
            pltpu.SemaphoreType.DMA,
        ],
    )(_gather_body)
    return k(x, ssrc, sdst)


def _stats_body(m_hbm, dst_hbm, s1_hbm, s2_hbm, mx_hbm, mn_hbm,
                rec_hbm, rid_hbm,
                m_v, d_v, st_s, st_q, st_x, st_n, ids_v, rec_v, rid_v,
                sem1, sem2, sem3, sem4):
    wid = _wid()
    start = wid * PT
    lanes = lax.iota(jnp.int32, 16)
    dump = jnp.full((16,), N, jnp.int32) + lanes

    for t in range(NG):
        rec_v[pl.ds(0 * D + t * 16, 16)] = jnp.zeros((16,), jnp.float32)
        rec_v[pl.ds(1 * D + t * 16, 16)] = jnp.zeros((16,), jnp.float32)
        rec_v[pl.ds(2 * D + t * 16, 16)] = jnp.full((16,), -jnp.inf, jnp.float32)
        rec_v[pl.ds(3 * D + t * 16, 16)] = jnp.full((16,), jnp.inf, jnp.float32)

    def edge_body(j, carry, base):
        (cur, own, p, ids, rid) = carry[:5]
        accs = carry[5]
        d = d_v[pl.ds(j, 16)][0]
        is_new = d != cur
        real = jnp.logical_and(is_new, cur >= 0)
        flush = jnp.logical_and(real, own)
        head = jnp.logical_and(real, jnp.logical_not(own))

        @pl.when(flush)
        def _stage():
            for t in range(NG):
                st_s[pl.ds(p * D + t * 16, 16)] = accs[0][t]
                st_q[pl.ds(p * D + t * 16, 16)] = accs[1][t]
                st_x[pl.ds(p * D + t * 16, 16)] = accs[2][t]
                st_n[pl.ds(p * D + t * 16, 16)] = accs[3][t]

        @pl.when(head)
        def _head_rec():
            for t in range(NG):
                rec_v[pl.ds(0 * D + t * 16, 16)] = accs[0][t]
                rec_v[pl.ds(1 * D + t * 16, 16)] = accs[1][t]
                rec_v[pl.ds(2 * D + t * 16, 16)] = accs[2][t]
                rec_v[pl.ds(3 * D + t * 16, 16)] = accs[3][t]

        rid2 = jnp.where(head, jnp.where(lanes == 0, cur, rid), rid)
        ids2 = jnp.where(flush, jnp.where(lanes == p, cur, ids), ids)
        p2 = jnp.where(flush, p + 1, p)
        do_drain = jnp.logical_and(flush, p2 == K)

        @pl.when(do_drain)
        def _drain():
            ids_v[pl.ds(0, 16)] = ids2
            hs = []
            for r in range(K):
                idr = ids_v[pl.ds(r, 16)][0]
                rsl = pl.ds(r * D, D)
                hs.append(pltpu.async_copy(st_s.at[rsl], s1_hbm.at[idr], sem1))
                hs.append(pltpu.async_copy(st_q.at[rsl], s2_hbm.at[idr], sem2))
                hs.append(pltpu.async_copy(st_x.at[rsl], mx_hbm.at[idr], sem3))
                hs.append(pltpu.async_copy(st_n.at[rsl], mn_hbm.at[idr], sem4))
            for h in hs:
                h.wait()

        p3 = jnp.where(do_drain, 0, p2)
        ids3 = jnp.where(do_drain, dump, ids2)
        own2 = jnp.logical_or(own, real)
        cur2 = jnp.where(is_new, d, cur)

        new_accs = ([], [], [], [])
        for t in range(NG):
            v = m_v[pl.ds(j * D + t * 16, 16)]
            q = v * v
            new_accs[0].append(jnp.where(is_new, v, accs[0][t] + v))
            new_accs[1].append(jnp.where(is_new, q, accs[1][t] + q))
            new_accs[2].append(jnp.where(is_new, v, jnp.maximum(accs[2][t], v)))
            new_accs[3].append(jnp.where(is_new, v, jnp.minimum(accs[3][t], v)))
        return (cur2, own2, p3, ids3, rid2, new_accs)

    def chunk_body(k, carry):
        base = start + k * CH
        pltpu.sync_copy(m_hbm.at[pl.ds(base * D, CH * D)], m_v)
        pltpu.sync_copy(dst_hbm.at[pl.ds(base, CH)], d_v.at[pl.ds(0, CH)])
        return lax.fori_loop(0, CH, lambda j, c: edge_body(j, c, base), carry)

    zero = jnp.zeros((16,), jnp.float32)
    accs0 = ([zero] * NG, [zero] * NG, [zero] * NG, [zero] * NG)
    init = (jnp.int32(-1), jnp.bool_(False), jnp.int32(0), dump, dump, accs0)
    carry = lax.fori_loop(0, PT // CH, chunk_body, init)
    (final_cur, final_own, final_p, final_ids, final_rid, final_accs) = carry

    for t in range(NG):
        rec_v[pl.ds(4 * D + t * 16, 16)] = final_accs[0][t]
        rec_v[pl.ds(5 * D + t * 16, 16)] = final_accs[1][t]
        rec_v[pl.ds(6 * D + t * 16, 16)] = final_accs[2][t]
        rec_v[pl.ds(7 * D + t * 16, 16)] = final_accs[3][t]
    rid_f = jnp.where(jnp.logical_not(final_own),
                      jnp.where(lanes == 0, final_cur, final_rid), final_rid)
    rid_f = jnp.where(lanes == 1, final_cur, rid_f)
    rid_v[...] = rid_f
    pltpu.sync_copy(rec_v, rec_hbm.at[pl.ds(wid * 8 * D, 8 * D)])
    pltpu.sync_copy(rid_v, rid_hbm.at[pl.ds(wid * 16, 16)])

    ids_v[pl.ds(0, 16)] = final_ids
    hs = []
    for r in range(K):
        idr = ids_v[pl.ds(r, 16)][0]
        rsl = pl.ds(r * D, D)
        hs.append(pltpu.async_copy(st_s.at[rsl], s1_hbm.at[idr], sem1))
        hs.append(pltpu.async_copy(st_q.at[rsl], s2_hbm.at[idr], sem2))
        hs.append(pltpu.async_copy(st_x.at[rsl], mx_hbm.at[idr], sem3))
        hs.append(pltpu.async_copy(st_n.at[rsl], mn_hbm.at[idr], sem4))
    for h in hs:
        h.wait()


def _stats_sc(m, sdst):
    k = functools.partial(
        pl.kernel,
        mesh=_mesh(),
        out_type=[jax.ShapeDtypeStruct((NPAD, D), jnp.float32)
                  for _ in range(4)]
        + [jax.ShapeDtypeStruct((NW * 8 * D,), jnp.float32),
           jax.ShapeDtypeStruct((NW * 16,), jnp.int32)],
        scratch_types=[
            pltpu.VMEM((CH * D,), jnp.float32),
            pltpu.VMEM((CH + 16,), jnp.int32),
            pltpu.VMEM((K * D,), jnp.float32),
            pltpu.VMEM((K * D,), jnp.float32),
            pltpu.VMEM((K * D,), jnp.float32),
            pltpu.VMEM((K * D,), jnp.float32),
            pltpu.VMEM((32,), jnp.int32),
            pltpu.VMEM((8 * D,), jnp.float32),
            pltpu.VMEM((16,), jnp.int32),
            pltpu.SemaphoreType.DMA,
            pltpu.SemaphoreType.DMA,
            pltpu.SemaphoreType.DMA,
            pltpu.SemaphoreType.DMA,
        ],
    )(_stats_body)
    return k(m.reshape(E * D), sdst)


NR = 2 * NW
RPT = NPAD // NW


def _merge_body(rec_hbm, rid_hbm, s1i, s2i, mxi, mni,
                s1o, s2o, mxo, mno,
                rec_v, rid_v, mg_s, mg_q, mg_x, mg_n, mid_v, buf_v):
    wid = _wid()
    lanes = lax.iota(jnp.int32, 16)
    pltpu.sync_copy(rec_hbm, rec_v)
    pltpu.sync_copy(rid_hbm, rid_v.at[pl.ds(0, NR * 8)])

    def rec_body(r, carry):
        (cur, q, bank) = carry[:3]
        accs = carry[3]
        trow = (r // 2) * 16 + (r % 2)
        idr = rid_v[pl.ds(trow, 16)][0]
        is_new = idr != cur
        flushq = jnp.logical_and(is_new, cur >= 0)

        @pl.when(flushq)
        def _fl():
            for t in range(NG):
                mg_s[pl.ds(q * D + t * 16, 16)] = accs[0][t]
                mg_q[pl.ds(q * D + t * 16, 16)] = accs[1][t]
                mg_x[pl.ds(q * D + t * 16, 16)] = accs[2][t]
                mg_n[pl.ds(q * D + t * 16, 16)] = accs[3][t]

        bank2 = jnp.where(flushq, jnp.where(lanes == q % 16, cur, bank), bank)
        q2 = jnp.where(flushq, q + 1, q)
        bfull = jnp.logical_and(flushq, q2 % 16 == 0)

        @pl.when(bfull)
        def _bank():
            mid_v[pl.ds(q2 - 16, 16)] = bank2

        bank3 = jnp.where(bfull, jnp.full((16,), N, jnp.int32), bank2)
        cur2 = jnp.where(is_new, idr, cur)
        row = (r // 2) * 8 + (r % 2) * 4
        new_accs = ([], [], [], [])
        for t in range(NG):
            vs = rec_v[pl.ds(row * D + t * 16, 16)]
            vq = rec_v[pl.ds((row + 1) * D + t * 16, 16)]
            vx = rec_v[pl.ds((row + 2) * D + t * 16, 16)]
            vn = rec_v[pl.ds((row + 3) * D + t * 16, 16)]
            new_accs[0].append(jnp.where(is_new, vs, accs[0][t] + vs))
            new_accs[1].append(jnp.where(is_new, vq, accs[1][t] + vq))
            new_accs[2].append(jnp.where(is_new, vx,
                                         jnp.maximum(accs[2][t], vx)))
            new_accs[3].append(jnp.where(is_new, vn,
                                         jnp.minimum(accs[3][t], vn)))
        return (cur2, q2, bank3, new_accs)

    zero = jnp.zeros((16,), jnp.float32)
    accs0 = ([zero] * NG, [zero] * NG, [zero] * NG, [zero] * NG)
    init = (jnp.int32(-1), jnp.int32(0), jnp.full((16,), N, jnp.int32), accs0)
    (cur_f, q_f, bank_f, accs_f) = lax.fori_loop(0, NR, rec_body, init)

    for t in range(NG):
        mg_s[pl.ds(q_f * D + t * 16, 16)] = accs_f[0][t]
        mg_q[pl.ds(q_f * D + t * 16, 16)] = accs_f[1][t]
        mg_x[pl.ds(q_f * D + t * 16, 16)] = accs_f[2][t]
        mg_n[pl.ds(q_f * D + t * 16, 16)] = accs_f[3][t]
    bank_l = jnp.where(lanes == q_f % 16, cur_f, bank_f)
    mid_v[pl.ds((q_f // 16) * 16, 16)] = bank_l
    q_n = q_f + 1

    lo = wid * RPT
    for (s_in, s_out, mg) in ((s1i, s1o, mg_s), (s2i, s2o, mg_q),
                              (mxi, mxo, mg_x), (mni, mno, mg_n)):
        pltpu.sync_copy(s_in.at[pl.ds(lo * D, RPT * D)], buf_v)

        def ov_body(s, carry, mg=mg):
            mid = mid_v[pl.ds(s, 16)][0]
            hit = jnp.logical_and(
                s < q_n,
                jnp.logical_and(mid >= lo, mid < lo + RPT))

            @pl.when(hit)
            def _ov():
                for t in range(NG):
                    buf_v[pl.ds((mid - lo) * D + t * 16, 16)] = (
                        mg[pl.ds(s * D + t * 16, 16)])

            return carry

        lax.fori_loop(0, NR + 1, ov_body, 0)
        pltpu.sync_copy(buf_v, s_out.at[pl.ds(lo * D, RPT * D)])


def _merge_sc(rec, rid, s1, s2, mx, mn):
    k = functools.partial(
        pl.kernel,
        mesh=_mesh(),
        out_type=[jax.ShapeDtypeStruct((NPAD * D,), jnp.float32)
                  for _ in range(4)],
        scratch_types=[
            pltpu.VMEM((NR * 4 * D,), jnp.float32),
            pltpu.VMEM((NR * 8 + 16,), jnp.int32),
            pltpu.VMEM(((NR + 16) * D,), jnp.float32),
            pltpu.VMEM(((NR + 16) * D,), jnp.float32),
            pltpu.VMEM(((NR + 16) * D,), jnp.float32),
            pltpu.VMEM(((NR + 16) * D,), jnp.float32),
            pltpu.VMEM((NR + 32,), jnp.int32),
            pltpu.VMEM((RPT * D,), jnp.float32),
        ],
    )(_merge_body)
    out = k(rec, rid, s1.reshape(NPAD * D), s2.reshape(NPAD * D),
            mx.reshape(NPAD * D), mn.reshape(NPAD * D))
    return [o.reshape(NPAD, D) for o in out]


def _edge_mlp_body(hs_ref, hd_ref, wa_ref, wb_ref, b_ref, o_ref):
    acc = jnp.dot(hs_ref[...], wa_ref[...], preferred_element_type=jnp.float32)
    acc = acc + jnp.dot(hd_ref[...], wb_ref[...], preferred_element_type=jnp.float32)
    o_ref[...] = jnp.maximum(acc + b_ref[...], 0.0)


def _edge_mlp(hs, hd, w_pre, b_pre):
    return pl.pallas_call(
        _edge_mlp_body,
        grid=(E // EB,),
        in_specs=[
            pl.BlockSpec((EB, D), lambda i: (i, 0)),
            pl.BlockSpec((EB, D), lambda i: (i, 0)),
            pl.BlockSpec((D, D), lambda i: (0, 0)),
            pl.BlockSpec((D, D), lambda i: (0, 0)),
            pl.BlockSpec((1, D), lambda i: (0, 0)),
        ],
        out_specs=pl.BlockSpec((EB, D), lambda i: (i, 0)),
        out_shape=jax.ShapeDtypeStruct((E, D), jnp.float32),
    )(hs, hd, w_pre[:D], w_pre[D:], b_pre.reshape(1, D))


def _embed_body(h_ref, w_ref, b_ref, o_ref):
    o_ref[...] = (
        jnp.dot(h_ref[...], w_ref[...], preferred_element_type=jnp.float32)
        + b_ref[...]
    )


def _embed(h, w_h, b_h):
    return pl.pallas_call(
        _embed_body,
        grid=(N // NB,),
        in_specs=[
            pl.BlockSpec((NB, D), lambda i: (i, 0)),
            pl.BlockSpec((D, D), lambda i: (0, 0)),
            pl.BlockSpec((1, D), lambda i: (0, 0)),
        ],
        out_specs=pl.BlockSpec((NB, D), lambda i: (i, 0)),
        out_shape=jax.ShapeDtypeStruct((N, D), jnp.float32),
    )(h, w_h, b_h.reshape(1, D))


def _post_a_body(x_ref, s1_ref, s2_ref, mx_ref, mn_ref, c0_ref, c1_ref,
                 sn_ref, w_ref, b_ref, y_ref, cs_ref, css_ref):
    cnt = c0_ref[...] + c1_ref[...]
    d = jnp.maximum(cnt, 1.0)
    inv_d = 1.0 / d
    has = cnt > 0.0
    mean = jnp.where(has, s1_ref[...] * inv_d, 0.0)
    var = jnp.where(has,
                    jnp.maximum(s2_ref[...] * inv_d - mean * mean, 0.0), 0.0)
    std = jnp.sqrt(var + 1e-5)
    mx = jnp.where(has, mx_ref[...], 0.0)
    mn = jnp.where(has, mn_ref[...], 0.0)
    logd = jnp.log(d + 1.0)
    amp = logd * (1.0 / AVG_D_LOG)
    att = AVG_D_LOG / logd
    w = w_ref[...]

    acc = jnp.dot(x_ref[...], w[0:D], preferred_element_type=jnp.float32)
    acc_a = jnp.zeros_like(acc)
    acc_t = jnp.zeros_like(acc)
    stats = (mean, mx, mn, std)
    for k in range(4):
        s = stats[k]
        acc = acc + jnp.dot(s, w[D + k * D:D + (k + 1) * D],
                            preferred_element_type=jnp.float32)
        acc_a = acc_a + jnp.dot(s, w[5 * D + k * D:5 * D + (k + 1) * D],
                                preferred_element_type=jnp.float32)
        acc_t = acc_t + jnp.dot(s, w[9 * D + k * D:9 * D + (k + 1) * D],
                                preferred_element_type=jnp.float32)
    y = (acc + amp * acc_a + att * acc_t + b_ref[...]) * sn_ref[...]
    y_ref[...] = y

    @pl.when(pl.program_id(0) == 0)
    def _init():
        cs_ref[...] = jnp.zeros_like(cs_ref)
        css_ref[...] = jnp.zeros_like(css_ref)

    cs_ref[...] += jnp.sum(y, axis=0, keepdims=True)
    css_ref[...] += jnp.sum(y * y, axis=0, keepdims=True)


def _post_b_body(x_ref, y_ref, cs_ref, css_ref, o_ref):
    mu = cs_ref[...] * (1.0 / N)
    vv = css_ref[...] * (1.0 / N) - mu * mu
    yn = (y_ref[...] - mu) * jax.lax.rsqrt(vv + 1e-5)
    o_ref[...] = x_ref[...] + jnp.maximum(yn, 0.0)


def _post(x, s1, s2, mx, mn, c0, c1, snorm_n, w_post, b_post):
    grid = (N // NB,)
    nspec = pl.BlockSpec((NB, D), lambda i: (i, 0))
    one_spec = pl.BlockSpec((NB, 1), lambda i: (i, 0))
    col_spec = pl.BlockSpec((1, D), lambda i: (0, 0))
    y, cs, css = pl.pallas_call(
        _post_a_body,
        grid=grid,
        in_specs=[nspec, nspec, nspec, nspec, nspec, one_spec, one_spec,
                  one_spec, pl.BlockSpec((13 * D, D), lambda i: (0, 0)),
                  col_spec],
        out_specs=[nspec, col_spec, col_spec],
        out_shape=[
            jax.ShapeDtypeStruct((N, D), jnp.float32),
            jax.ShapeDtypeStruct((1, D), jnp.float32),
            jax.ShapeDtypeStruct((1, D), jnp.float32),
        ],
    )(x, s1, s2, mx, mn, c0, c1, snorm_n, w_post, b_post.reshape(1, D))
    return pl.pallas_call(
        _post_b_body,
        grid=grid,
        in_specs=[nspec, nspec, col_spec, col_spec],
        out_specs=nspec,
        out_shape=jax.ShapeDtypeStruct((N, D), jnp.float32),
    )(x, y, cs, css)


def kernel(h, edge_index, e, snorm_n, snorm_e, W_h, b_h, W_pre, b_pre,
           W_post, b_post):
    src = edge_index[0].astype(jnp.int32)
    dst = edge_index[1].astype(jnp.int32)
    perm = jnp.argsort(dst)
    sdst = dst[perm]
    ssrc = src[perm]
    cnt2 = _cnt_sc(sdst)
    c0 = cnt2[0].reshape(N, 1)
    c1 = cnt2[1].reshape(N, 1)
    x = _embed(h, W_h, b_h)
    for l in range(L):
        hs, hd = _gather_sc(x, ssrc, sdst)
        m = _edge_mlp(hs, hd, W_pre[l], b_pre[l])
        s1p, s2p, mxp, mnp, rec, rid = _stats_sc(m, sdst)
        s1, s2, mx, mn = _merge_sc(rec, rid, s1p, s2p, mxp, mnp)
        x = _post(x, s1, s2, mx, mn, c0, c1, snorm_n, W_post[l], b_post[l])
    return x


# speedup vs baseline: 2.8041x; 1.0462x over previous
import functools

import jax
import jax.numpy as jnp
from jax import lax
from jax.experimental import pallas as pl
from jax.experimental.pallas import tpu as pltpu
from jax.experimental.pallas import tpu_sc as plsc

N = 10000
E = 320000
D = 128
L = 4
AVG_D_LOG = 3.4965

NW = 32
PT = E // NW
EB = 2560
NB = 2000
GC = 80
CC = 80
CH = 200
K = 16
NPAD = 10240
NG = 8


def _mesh():
    return plsc.VectorSubcoreMesh(core_axis_name="c", subcore_axis_name="s")


def _wid():
    return lax.axis_index("s") * 2 + lax.axis_index("c")


def _cnt_body(dst_hbm, cnt_hbm, di_v, ones_v, z_v, acc_sh):
    sid = lax.axis_index("s")
    cid = lax.axis_index("c")
    wid = sid * 2 + cid

    def fill(i, carry):
        ones_v[pl.ds(i * 16, 16)] = jnp.full((16,), 1.0, jnp.float32)
        return carry

    lax.fori_loop(0, CC // 16, fill, 0)

    def fillz(i, carry):
        z_v[pl.ds(i * 16, 16)] = jnp.zeros((16,), jnp.float32)
        return carry

    lax.fori_loop(0, 2000 // 16, fillz, 0)

    @pl.when(sid == 0)
    def _zero():
        def zc(i, carry):
            pltpu.sync_copy(z_v, acc_sh.at[pl.ds(i * 2000, 2000)])
            return carry

        lax.fori_loop(0, N // 2000, zc, 0)

    plsc.subcore_barrier()

    start = wid * PT

    def chunk(i, carry):
        pltpu.sync_copy(dst_hbm.at[pl.ds(start + i * CC, CC)], di_v)
        pltpu.sync_copy(ones_v, acc_sh.at[di_v], add=True)
        return carry

    lax.fori_loop(0, PT // CC, chunk, 0)
    plsc.subcore_barrier()

    @pl.when(sid == 0)
    def _export():
        pltpu.sync_copy(acc_sh, cnt_hbm.at[cid])


def _cnt_sc(sdst):
    k = functools.partial(
        pl.kernel,
        mesh=_mesh(),
        out_type=jax.ShapeDtypeStruct((2, N), jnp.float32),
        scratch_types=[
            pltpu.VMEM((CC,), jnp.int32),
            pltpu.VMEM((CC,), jnp.float32),
            pltpu.VMEM((2000,), jnp.float32),
            pltpu.VMEM_SHARED((N,), jnp.float32),
        ],
    )(_cnt_body)
    return k(sdst)


GB = 400


def _gather_body(x_hbm, src_hbm, dst_hbm, hs_hbm, hd_hbm,
                 si_v, di_v, hs_v, hd_v, sg, sw):
    start = _wid() * PT
    nch = PT // GB

    def chunk(i, carry):
        base = start + i * GB

        @pl.when(i > 0)
        def _w():
            prev = start + (i - 1) * GB
            pltpu.make_async_copy(
                hs_v, hs_hbm.at[pl.ds(prev, GB)], sw).wait()
            pltpu.make_async_copy(
                hd_v, hd_hbm.at[pl.ds(prev, GB)], sw).wait()

        pltpu.sync_copy(src_hbm.at[pl.ds(base, GB)], si_v)
        pltpu.sync_copy(dst_hbm.at[pl.ds(base, GB)], di_v)
        hs = []
        for g in range(GB // 80):
            sl = pl.ds(g * 80, 80)
            hs.append(pltpu.async_copy(
                x_hbm.at[si_v.at[sl]], hs_v.at[sl], sg))
            hs.append(pltpu.async_copy(
                x_hbm.at[di_v.at[sl]], hd_v.at[sl], sg))
        for h in hs:
            h.wait()
        pltpu.async_copy(hs_v, hs_hbm.at[pl.ds(base, GB)], sw)
        pltpu.async_copy(hd_v, hd_hbm.at[pl.ds(base, GB)], sw)
        return carry

    lax.fori_loop(0, nch, chunk, 0)
    last = start + (nch - 1) * GB
    pltpu.make_async_copy(hs_v, hs_hbm.at[pl.ds(last, GB)], sw).wait()
    pltpu.make_async_copy(hd_v, hd_hbm.at[pl.ds(last, GB)], sw).wait()


def _gather_sc(x, ssrc, sdst):
    k = functools.partial(
        pl.kernel,
        mesh=_mesh(),
        out_type=[jax.ShapeDtypeStruct((E, D), jnp.float32),
                  jax.ShapeDtypeStruct((E, D), jnp.float32)],
        scratch_types=[
            pltpu.VMEM((GB,), jnp.int32),
            pltpu.VMEM((GB,), jnp.int32),
            pltpu.VMEM((GB, D), jnp.float32),
            pltpu.VMEM((GB, D), jnp.float32),
            pltpu.SemaphoreType.DMA,
            pltpu.SemaphoreType.DMA,
        ],
    )(_gather_body)
    return k(x, ssrc, sdst)


def _stats_body(m_hbm, dst_hbm, s1_hbm, s2_hbm, mx_hbm, mn_hbm,
                rec_hbm, rid_hbm,
                m_v0, m_v1, d_v0, d_v1, st_s, st_q, st_x, st_n, ids_v,
                rec_v, rid_v, sem1, sem2, sem3, sem4, sl0, sl1):
    wid = _wid()
    start = wid * PT
    nch = PT // CH
    m_bufs = (m_v0, m_v1)
    d_bufs = (d_v0, d_v1)
    lsems = (sl0, sl1)

    def load_chunk(c, b):
        base = start + c * CH
        pltpu.async_copy(m_hbm.at[pl.ds(base * D, CH * D)], m_bufs[b],
                         lsems[b])
        pltpu.async_copy(dst_hbm.at[pl.ds(base, CH)],
                         d_bufs[b].at[pl.ds(0, CH)], lsems[b])

    def wait_chunk(c, b):
        base = start + c * CH
        pltpu.make_async_copy(m_hbm.at[pl.ds(base * D, CH * D)], m_bufs[b],
                              lsems[b]).wait()
        pltpu.make_async_copy(dst_hbm.at[pl.ds(base, CH)],
                              d_bufs[b].at[pl.ds(0, CH)], lsems[b]).wait()
    lanes = lax.iota(jnp.int32, 16)
    dump = jnp.full((16,), N, jnp.int32) + lanes

    for t in range(NG):
        rec_v[pl.ds(0 * D + t * 16, 16)] = jnp.zeros((16,), jnp.float32)
        rec_v[pl.ds(1 * D + t * 16, 16)] = jnp.zeros((16,), jnp.float32)
        rec_v[pl.ds(2 * D + t * 16, 16)] = jnp.full((16,), -jnp.inf, jnp.float32)
        rec_v[pl.ds(3 * D + t * 16, 16)] = jnp.full((16,), jnp.inf, jnp.float32)

    def edge_body(j, carry, base, m_v, d_v):
        (cur, own, p, ids, rid) = carry[:5]
        accs = carry[5]
        d = d_v[pl.ds(j, 16)][0]
        is_new = d != cur
        real = jnp.logical_and(is_new, cur >= 0)
        flush = jnp.logical_and(real, own)
        head = jnp.logical_and(real, jnp.logical_not(own))

        @pl.when(flush)
        def _stage():
            for t in range(NG):
                st_s[pl.ds(p * D + t * 16, 16)] = accs[0][t]
                st_q[pl.ds(p * D + t * 16, 16)] = accs[1][t]
                st_x[pl.ds(p * D + t * 16, 16)] = accs[2][t]
                st_n[pl.ds(p * D + t * 16, 16)] = accs[3][t]

        @pl.when(head)
        def _head_rec():
            for t in range(NG):
                rec_v[pl.ds(0 * D + t * 16, 16)] = accs[0][t]
                rec_v[pl.ds(1 * D + t * 16, 16)] = accs[1][t]
                rec_v[pl.ds(2 * D + t * 16, 16)] = accs[2][t]
                rec_v[pl.ds(3 * D + t * 16, 16)] = accs[3][t]

        rid2 = jnp.where(head, jnp.where(lanes == 0, cur, rid), rid)
        ids2 = jnp.where(flush, jnp.where(lanes == p, cur, ids), ids)
        p2 = jnp.where(flush, p + 1, p)
        do_drain = jnp.logical_and(flush, p2 == K)

        @pl.when(do_drain)
        def _drain():
            ids_v[pl.ds(0, 16)] = ids2
            hs = []
            for r in range(K):
                idr = ids_v[pl.ds(r, 16)][0]
                rsl = pl.ds(r * D, D)
                hs.append(pltpu.async_copy(st_s.at[rsl], s1_hbm.at[idr], sem1))
                hs.append(pltpu.async_copy(st_q.at[rsl], s2_hbm.at[idr], sem2))
                hs.append(pltpu.async_copy(st_x.at[rsl], mx_hbm.at[idr], sem3))
                hs.append(pltpu.async_copy(st_n.at[rsl], mn_hbm.at[idr], sem4))
            for h in hs:
                h.wait()

        p3 = jnp.where(do_drain, 0, p2)
        ids3 = jnp.where(do_drain, dump, ids2)
        own2 = jnp.logical_or(own, real)
        cur2 = jnp.where(is_new, d, cur)

        new_accs = ([], [], [], [])
        for t in range(NG):
            v = m_v[pl.ds(j * D + t * 16, 16)]
            q = v * v
            new_accs[0].append(jnp.where(is_new, v, accs[0][t] + v))
            new_accs[1].append(jnp.where(is_new, q, accs[1][t] + q))
            new_accs[2].append(jnp.where(is_new, v, jnp.maximum(accs[2][t], v)))
            new_accs[3].append(jnp.where(is_new, v, jnp.minimum(accs[3][t], v)))
        return (cur2, own2, p3, ids3, rid2, new_accs)

    def chunk_pair(k2, carry):
        for b in range(2):
            c = k2 * 2 + b
            base = start + c * CH
            wait_chunk(c, b)

            @pl.when(c + 1 < nch)
            def _pref(c=c, b=b):
                load_chunk(c + 1, 1 - b)

            carry = lax.fori_loop(
                0, CH,
                lambda j, cr, base=base, b=b: edge_body(
                    j, cr, base, m_bufs[b], d_bufs[b]),
                carry)
        return carry

    zero = jnp.zeros((16,), jnp.float32)
    accs0 = ([zero] * NG, [zero] * NG, [zero] * NG, [zero] * NG)
    init = (jnp.int32(-1), jnp.bool_(False), jnp.int32(0), dump, dump, accs0)
    load_chunk(0, 0)
    carry = lax.fori_loop(0, nch // 2, chunk_pair, init)
    (final_cur, final_own, final_p, final_ids, final_rid, final_accs) = carry

    for t in range(NG):
        rec_v[pl.ds(4 * D + t * 16, 16)] = final_accs[0][t]
        rec_v[pl.ds(5 * D + t * 16, 16)] = final_accs[1][t]
        rec_v[pl.ds(6 * D + t * 16, 16)] = final_accs[2][t]
        rec_v[pl.ds(7 * D + t * 16, 16)] = final_accs[3][t]
    rid_f = jnp.where(jnp.logical_not(final_own),
                      jnp.where(lanes == 0, final_cur, final_rid), final_rid)
    rid_f = jnp.where(lanes == 1, final_cur, rid_f)
    rid_v[...] = rid_f
    pltpu.sync_copy(rec_v, rec_hbm.at[pl.ds(wid * 8 * D, 8 * D)])
    pltpu.sync_copy(rid_v, rid_hbm.at[pl.ds(wid * 16, 16)])

    ids_v[pl.ds(0, 16)] = final_ids
    hs = []
    for r in range(K):
        idr = ids_v[pl.ds(r, 16)][0]
        rsl = pl.ds(r * D, D)
        hs.append(pltpu.async_copy(st_s.at[rsl], s1_hbm.at[idr], sem1))
        hs.append(pltpu.async_copy(st_q.at[rsl], s2_hbm.at[idr], sem2))
        hs.append(pltpu.async_copy(st_x.at[rsl], mx_hbm.at[idr], sem3))
        hs.append(pltpu.async_copy(st_n.at[rsl], mn_hbm.at[idr], sem4))
    for h in hs:
        h.wait()


def _stats_sc(m, sdst):
    k = functools.partial(
        pl.kernel,
        mesh=_mesh(),
        out_type=[jax.ShapeDtypeStruct((NPAD, D), jnp.float32)
                  for _ in range(4)]
        + [jax.ShapeDtypeStruct((NW * 8 * D,), jnp.float32),
           jax.ShapeDtypeStruct((NW * 16,), jnp.int32)],
        scratch_types=[
            pltpu.VMEM((CH * D,), jnp.float32),
            pltpu.VMEM((CH * D,), jnp.float32),
            pltpu.VMEM((CH + 16,), jnp.int32),
            pltpu.VMEM((CH + 16,), jnp.int32),
            pltpu.VMEM((K * D,), jnp.float32),
            pltpu.VMEM((K * D,), jnp.float32),
            pltpu.VMEM((K * D,), jnp.float32),
            pltpu.VMEM((K * D,), jnp.float32),
            pltpu.VMEM((32,), jnp.int32),
            pltpu.VMEM((8 * D,), jnp.float32),
            pltpu.VMEM((16,), jnp.int32),
            pltpu.SemaphoreType.DMA,
            pltpu.SemaphoreType.DMA,
            pltpu.SemaphoreType.DMA,
            pltpu.SemaphoreType.DMA,
            pltpu.SemaphoreType.DMA,
            pltpu.SemaphoreType.DMA,
        ],
    )(_stats_body)
    return k(m.reshape(E * D), sdst)


NR = 2 * NW
RPT = NPAD // NW


def _merge_body(rec_hbm, rid_hbm, s1i, s2i, mxi, mni,
                s1o, s2o, mxo, mno,
                rec_v, rid_v, mg_s, mg_q, mg_x, mg_n, mid_v, buf_v):
    wid = _wid()
    lanes = lax.iota(jnp.int32, 16)
    pltpu.sync_copy(rec_hbm, rec_v)
    pltpu.sync_copy(rid_hbm, rid_v.at[pl.ds(0, NR * 8)])

    def rec_body(r, carry):
        (cur, q, bank) = carry[:3]
        accs = carry[3]
        trow = (r // 2) * 16 + (r % 2)
        idr = rid_v[pl.ds(trow, 16)][0]
        is_new = idr != cur
        flushq = jnp.logical_and(is_new, cur >= 0)

        @pl.when(flushq)
        def _fl():
            for t in range(NG):
                mg_s[pl.ds(q * D + t * 16, 16)] = accs[0][t]
                mg_q[pl.ds(q * D + t * 16, 16)] = accs[1][t]
                mg_x[pl.ds(q * D + t * 16, 16)] = accs[2][t]
                mg_n[pl.ds(q * D + t * 16, 16)] = accs[3][t]

        bank2 = jnp.where(flushq, jnp.where(lanes == q % 16, cur, bank), bank)
        q2 = jnp.where(flushq, q + 1, q)
        bfull = jnp.logical_and(flushq, q2 % 16 == 0)

        @pl.when(bfull)
        def _bank():
            mid_v[pl.ds(q2 - 16, 16)] = bank2

        bank3 = jnp.where(bfull, jnp.full((16,), N, jnp.int32), bank2)
        cur2 = jnp.where(is_new, idr, cur)
        row = (r // 2) * 8 + (r % 2) * 4
        new_accs = ([], [], [], [])
        for t in range(NG):
            vs = rec_v[pl.ds(row * D + t * 16, 16)]
            vq = rec_v[pl.ds((row + 1) * D + t * 16, 16)]
            vx = rec_v[pl.ds((row + 2) * D + t * 16, 16)]
            vn = rec_v[pl.ds((row + 3) * D + t * 16, 16)]
            new_accs[0].append(jnp.where(is_new, vs, accs[0][t] + vs))
            new_accs[1].append(jnp.where(is_new, vq, accs[1][t] + vq))
            new_accs[2].append(jnp.where(is_new, vx,
                                         jnp.maximum(accs[2][t], vx)))
            new_accs[3].append(jnp.where(is_new, vn,
                                         jnp.minimum(accs[3][t], vn)))
        return (cur2, q2, bank3, new_accs)

    zero = jnp.zeros((16,), jnp.float32)
    accs0 = ([zero] * NG, [zero] * NG, [zero] * NG, [zero] * NG)
    init = (jnp.int32(-1), jnp.int32(0), jnp.full((16,), N, jnp.int32), accs0)
    (cur_f, q_f, bank_f, accs_f) = lax.fori_loop(0, NR, rec_body, init)

    for t in range(NG):
        mg_s[pl.ds(q_f * D + t * 16, 16)] = accs_f[0][t]
        mg_q[pl.ds(q_f * D + t * 16, 16)] = accs_f[1][t]
        mg_x[pl.ds(q_f * D + t * 16, 16)] = accs_f[2][t]
        mg_n[pl.ds(q_f * D + t * 16, 16)] = accs_f[3][t]
    bank_l = jnp.where(lanes == q_f % 16, cur_f, bank_f)
    mid_v[pl.ds((q_f // 16) * 16, 16)] = bank_l
    q_n = q_f + 1

    lo = wid * RPT
    for (s_in, s_out, mg) in ((s1i, s1o, mg_s), (s2i, s2o, mg_q),
                              (mxi, mxo, mg_x), (mni, mno, mg_n)):
        pltpu.sync_copy(s_in.at[pl.ds(lo * D, RPT * D)], buf_v)

        def ov_body(s, carry, mg=mg):
            mid = mid_v[pl.ds(s, 16)][0]
            hit = jnp.logical_and(
                s < q_n,
                jnp.logical_and(mid >= lo, mid < lo + RPT))

            @pl.when(hit)
            def _ov():
                for t in range(NG):
                    buf_v[pl.ds((mid - lo) * D + t * 16, 16)] = (
                        mg[pl.ds(s * D + t * 16, 16)])

            return carry

        lax.fori_loop(0, NR + 1, ov_body, 0)
        pltpu.sync_copy(buf_v, s_out.at[pl.ds(lo * D, RPT * D)])


def _merge_sc(rec, rid, s1, s2, mx, mn):
    k = functools.partial(
        pl.kernel,
        mesh=_mesh(),
        out_type=[jax.ShapeDtypeStruct((NPAD * D,), jnp.float32)
                  for _ in range(4)],
        scratch_types=[
            pltpu.VMEM((NR * 4 * D,), jnp.float32),
            pltpu.VMEM((NR * 8 + 16,), jnp.int32),
            pltpu.VMEM(((NR + 16) * D,), jnp.float32),
            pltpu.VMEM(((NR + 16) * D,), jnp.float32),
            pltpu.VMEM(((NR + 16) * D,), jnp.float32),
            pltpu.VMEM(((NR + 16) * D,), jnp.float32),
            pltpu.VMEM((NR + 32,), jnp.int32),
            pltpu.VMEM((RPT * D,), jnp.float32),
        ],
    )(_merge_body)
    out = k(rec, rid, s1.reshape(NPAD * D), s2.reshape(NPAD * D),
            mx.reshape(NPAD * D), mn.reshape(NPAD * D))
    return [o.reshape(NPAD, D) for o in out]


def _edge_mlp_body(hs_ref, hd_ref, wa_ref, wb_ref, b_ref, o_ref):
    acc = jnp.dot(hs_ref[...], wa_ref[...], preferred_element_type=jnp.float32)
    acc = acc + jnp.dot(hd_ref[...], wb_ref[...], preferred_element_type=jnp.float32)
    o_ref[...] = jnp.maximum(acc + b_ref[...], 0.0)


def _edge_mlp(hs, hd, w_pre, b_pre):
    return pl.pallas_call(
        _edge_mlp_body,
        grid=(E // EB,),
        in_specs=[
            pl.BlockSpec((EB, D), lambda i: (i, 0)),
            pl.BlockSpec((EB, D), lambda i: (i, 0)),
            pl.BlockSpec((D, D), lambda i: (0, 0)),
            pl.BlockSpec((D, D), lambda i: (0, 0)),
            pl.BlockSpec((1, D), lambda i: (0, 0)),
        ],
        out_specs=pl.BlockSpec((EB, D), lambda i: (i, 0)),
        out_shape=jax.ShapeDtypeStruct((E, D), jnp.float32),
    )(hs, hd, w_pre[:D], w_pre[D:], b_pre.reshape(1, D))


def _embed_body(h_ref, w_ref, b_ref, o_ref):
    o_ref[...] = (
        jnp.dot(h_ref[...], w_ref[...], preferred_element_type=jnp.float32)
        + b_ref[...]
    )


def _embed(h, w_h, b_h):
    return pl.pallas_call(
        _embed_body,
        grid=(N // NB,),
        in_specs=[
            pl.BlockSpec((NB, D), lambda i: (i, 0)),
            pl.BlockSpec((D, D), lambda i: (0, 0)),
            pl.BlockSpec((1, D), lambda i: (0, 0)),
        ],
        out_specs=pl.BlockSpec((NB, D), lambda i: (i, 0)),
        out_shape=jax.ShapeDtypeStruct((N, D), jnp.float32),
    )(h, w_h, b_h.reshape(1, D))


def _post_a_body(x_ref, s1_ref, s2_ref, mx_ref, mn_ref, c0_ref, c1_ref,
                 sn_ref, w_ref, b_ref, y_ref, cs_ref, css_ref):
    cnt = c0_ref[...] + c1_ref[...]
    d = jnp.maximum(cnt, 1.0)
    inv_d = 1.0 / d
    has = cnt > 0.0
    mean = jnp.where(has, s1_ref[...] * inv_d, 0.0)
    var = jnp.where(has,
                    jnp.maximum(s2_ref[...] * inv_d - mean * mean, 0.0), 0.0)
    std = jnp.sqrt(var + 1e-5)
    mx = jnp.where(has, mx_ref[...], 0.0)
    mn = jnp.where(has, mn_ref[...], 0.0)
    logd = jnp.log(d + 1.0)
    amp = logd * (1.0 / AVG_D_LOG)
    att = AVG_D_LOG / logd
    w = w_ref[...]

    acc = jnp.dot(x_ref[...], w[0:D], preferred_element_type=jnp.float32)
    acc_a = jnp.zeros_like(acc)
    acc_t = jnp.zeros_like(acc)
    stats = (mean, mx, mn, std)
    for k in range(4):
        s = stats[k]
        acc = acc + jnp.dot(s, w[D + k * D:D + (k + 1) * D],
                            preferred_element_type=jnp.float32)
        acc_a = acc_a + jnp.dot(s, w[5 * D + k * D:5 * D + (k + 1) * D],
                                preferred_element_type=jnp.float32)
        acc_t = acc_t + jnp.dot(s, w[9 * D + k * D:9 * D + (k + 1) * D],
                                preferred_element_type=jnp.float32)
    y = (acc + amp * acc_a + att * acc_t + b_ref[...]) * sn_ref[...]
    y_ref[...] = y

    @pl.when(pl.program_id(0) == 0)
    def _init():
        cs_ref[...] = jnp.zeros_like(cs_ref)
        css_ref[...] = jnp.zeros_like(css_ref)

    cs_ref[...] += jnp.sum(y, axis=0, keepdims=True)
    css_ref[...] += jnp.sum(y * y, axis=0, keepdims=True)


def _post_b_body(x_ref, y_ref, cs_ref, css_ref, o_ref):
    mu = cs_ref[...] * (1.0 / N)
    vv = css_ref[...] * (1.0 / N) - mu * mu
    yn = (y_ref[...] - mu) * jax.lax.rsqrt(vv + 1e-5)
    o_ref[...] = x_ref[...] + jnp.maximum(yn, 0.0)


def _post(x, s1, s2, mx, mn, c0, c1, snorm_n, w_post, b_post):
    grid = (N // NB,)
    nspec = pl.BlockSpec((NB, D), lambda i: (i, 0))
    one_spec = pl.BlockSpec((NB, 1), lambda i: (i, 0))
    col_spec = pl.BlockSpec((1, D), lambda i: (0, 0))
    y, cs, css = pl.pallas_call(
        _post_a_body,
        grid=grid,
        in_specs=[nspec, nspec, nspec, nspec, nspec, one_spec, one_spec,
                  one_spec, pl.BlockSpec((13 * D, D), lambda i: (0, 0)),
                  col_spec],
        out_specs=[nspec, col_spec, col_spec],
        out_shape=[
            jax.ShapeDtypeStruct((N, D), jnp.float32),
            jax.ShapeDtypeStruct((1, D), jnp.float32),
            jax.ShapeDtypeStruct((1, D), jnp.float32),
        ],
    )(x, s1, s2, mx, mn, c0, c1, snorm_n, w_post, b_post.reshape(1, D))
    return pl.pallas_call(
        _post_b_body,
        grid=grid,
        in_specs=[nspec, nspec, col_spec, col_spec],
        out_specs=nspec,
        out_shape=jax.ShapeDtypeStruct((N, D), jnp.float32),
    )(x, y, cs, css)


def kernel(h, edge_index, e, snorm_n, snorm_e, W_h, b_h, W_pre, b_pre,
           W_post, b_post):
    src = edge_index[0].astype(jnp.int32)
    dst = edge_index[1].astype(jnp.int32)
    perm = jnp.argsort(dst)
    sdst = dst[perm]
    ssrc = src[perm]
    cnt2 = _cnt_sc(sdst)
    c0 = cnt2[0].reshape(N, 1)
    c1 = cnt2[1].reshape(N, 1)
    x = _embed(h, W_h, b_h)
    for l in range(L):
        hs, hd = _gather_sc(x, ssrc, sdst)
        m = _edge_mlp(hs, hd, W_pre[l], b_pre[l])
        s1p, s2p, mxp, mnp, rec, rid = _stats_sc(m, sdst)
        s1, s2, mx, mn = _merge_sc(rec, rid, s1p, s2p, mxp, mnp)
        x = _post(x, s1, s2, mx, mn, c0, c1, snorm_n, W_post[l], b_post[l])
    return x


# speedup vs baseline: 2.8618x; 1.0205x over previous
import functools

import jax
import jax.numpy as jnp
from jax import lax
from jax.experimental import pallas as pl
from jax.experimental.pallas import tpu as pltpu
from jax.experimental.pallas import tpu_sc as plsc

N = 10000
E = 320000
D = 128
L = 4
AVG_D_LOG = 3.4965

NW = 32
PT = E // NW
EB = 2560
NB = 2000
GC = 80
CC = 80
CH = 200
K = 16
NPAD = 10240
NG = 8


def _mesh():
    return plsc.VectorSubcoreMesh(core_axis_name="c", subcore_axis_name="s")


def _wid():
    return lax.axis_index("s") * 2 + lax.axis_index("c")


def _cnt_body(dst_hbm, cnt_hbm, di_v, ones_v, z_v, acc_sh):
    sid = lax.axis_index("s")
    cid = lax.axis_index("c")
    wid = sid * 2 + cid

    def fill(i, carry):
        ones_v[pl.ds(i * 16, 16)] = jnp.full((16,), 1.0, jnp.float32)
        return carry

    lax.fori_loop(0, CC // 16, fill, 0)

    def fillz(i, carry):
        z_v[pl.ds(i * 16, 16)] = jnp.zeros((16,), jnp.float32)
        return carry

    lax.fori_loop(0, 2000 // 16, fillz, 0)

    @pl.when(sid == 0)
    def _zero():
        def zc(i, carry):
            pltpu.sync_copy(z_v, acc_sh.at[pl.ds(i * 2000, 2000)])
            return carry

        lax.fori_loop(0, N // 2000, zc, 0)

    plsc.subcore_barrier()

    start = wid * PT

    def chunk(i, carry):
        pltpu.sync_copy(dst_hbm.at[pl.ds(start + i * CC, CC)], di_v)
        pltpu.sync_copy(ones_v, acc_sh.at[di_v], add=True)
        return carry

    lax.fori_loop(0, PT // CC, chunk, 0)
    plsc.subcore_barrier()

    @pl.when(sid == 0)
    def _export():
        pltpu.sync_copy(acc_sh, cnt_hbm.at[cid])


def _cnt_sc(sdst):
    k = functools.partial(
        pl.kernel,
        mesh=_mesh(),
        out_type=jax.ShapeDtypeStruct((2, N), jnp.float32),
        scratch_types=[
            pltpu.VMEM((CC,), jnp.int32),
            pltpu.VMEM((CC,), jnp.float32),
            pltpu.VMEM((2000,), jnp.float32),
            pltpu.VMEM_SHARED((N,), jnp.float32),
        ],
    )(_cnt_body)
    return k(sdst)


GB = 200


def _gather_body(x_hbm, src_hbm, dst_hbm, hs_hbm, hd_hbm,
                 si0, si1, di0, di1, hs0, hs1, hd0, hd1, sg0, sg1, sw):
    start = _wid() * PT
    nch = PT // GB
    sib = (si0, si1)
    dib = (di0, di1)
    hsb = (hs0, hs1)
    hdb = (hd0, hd1)
    sgb = (sg0, sg1)

    def load_idx(c, b):
        base = start + c * GB
        pltpu.async_copy(src_hbm.at[pl.ds(base, GB)], sib[b], sgb[b])
        pltpu.async_copy(dst_hbm.at[pl.ds(base, GB)], dib[b], sgb[b])

    def wait_idx(c, b):
        base = start + c * GB
        pltpu.make_async_copy(src_hbm.at[pl.ds(base, GB)], sib[b],
                              sgb[b]).wait()
        pltpu.make_async_copy(dst_hbm.at[pl.ds(base, GB)], dib[b],
                              sgb[b]).wait()

    load_idx(0, 0)

    def chunk(c2, carry):
        for b in range(2):
            c = c2 * 2 + b
            base = start + c * GB

            @pl.when(c >= 2)
            def _wprev(b=b, c=c):
                prev = start + (c - 2) * GB
                pltpu.make_async_copy(
                    hsb[b], hs_hbm.at[pl.ds(prev, GB)], sw).wait()
                pltpu.make_async_copy(
                    hdb[b], hd_hbm.at[pl.ds(prev, GB)], sw).wait()

            wait_idx(c, b)
            hs = []
            for (off, ln) in ((0, 80), (80, 80), (160, 40)):
                sl = pl.ds(off, ln)
                hs.append(pltpu.async_copy(
                    x_hbm.at[sib[b].at[sl]], hsb[b].at[sl], sgb[b]))
                hs.append(pltpu.async_copy(
                    x_hbm.at[dib[b].at[sl]], hdb[b].at[sl], sgb[b]))

            @pl.when(c + 1 < nch)
            def _pref(b=b, c=c):
                load_idx(c + 1, 1 - b)

            for h in hs:
                h.wait()
            pltpu.async_copy(hsb[b], hs_hbm.at[pl.ds(base, GB)], sw)
            pltpu.async_copy(hdb[b], hd_hbm.at[pl.ds(base, GB)], sw)
        return carry

    lax.fori_loop(0, nch // 2, chunk, 0)
    for c in (nch - 2, nch - 1):
        b = c % 2
        last = start + c * GB
        pltpu.make_async_copy(hsb[b], hs_hbm.at[pl.ds(last, GB)], sw).wait()
        pltpu.make_async_copy(hdb[b], hd_hbm.at[pl.ds(last, GB)], sw).wait()


def _gather_sc(x, ssrc, sdst):
    k = functools.partial(
        pl.kernel,
        mesh=_mesh(),
        out_type=[jax.ShapeDtypeStruct((E, D), jnp.float32),
                  jax.ShapeDtypeStruct((E, D), jnp.float32)],
        scratch_types=[
            pltpu.VMEM((GB,), jnp.int32),
            pltpu.VMEM((GB,), jnp.int32),
            pltpu.VMEM((GB,), jnp.int32),
            pltpu.VMEM((GB,), jnp.int32),
            pltpu.VMEM((GB, D), jnp.float32),
            pltpu.VMEM((GB, D), jnp.float32),
            pltpu.VMEM((GB, D), jnp.float32),
            pltpu.VMEM((GB, D), jnp.float32),
            pltpu.SemaphoreType.DMA,
            pltpu.SemaphoreType.DMA,
            pltpu.SemaphoreType.DMA,
        ],
    )(_gather_body)
    return k(x, ssrc, sdst)


def _stats_body(m_hbm, dst_hbm, s1_hbm, s2_hbm, mx_hbm, mn_hbm,
                rec_hbm, rid_hbm,
                m_v0, m_v1, d_v0, d_v1, st_s, st_q, st_x, st_n, ids_v,
                rec_v, rid_v, sem1, sem2, sem3, sem4, sl0, sl1):
    wid = _wid()
    start = wid * PT
    nch = PT // CH
    m_bufs = (m_v0, m_v1)
    d_bufs = (d_v0, d_v1)
    lsems = (sl0, sl1)

    def load_chunk(c, b):
        base = start + c * CH
        pltpu.async_copy(m_hbm.at[pl.ds(base * D, CH * D)], m_bufs[b],
                         lsems[b])
        pltpu.async_copy(dst_hbm.at[pl.ds(base, CH)],
                         d_bufs[b].at[pl.ds(0, CH)], lsems[b])

    def wait_chunk(c, b):
        base = start + c * CH
        pltpu.make_async_copy(m_hbm.at[pl.ds(base * D, CH * D)], m_bufs[b],
                              lsems[b]).wait()
        pltpu.make_async_copy(dst_hbm.at[pl.ds(base, CH)],
                              d_bufs[b].at[pl.ds(0, CH)], lsems[b]).wait()
    lanes = lax.iota(jnp.int32, 16)
    dump = jnp.full((16,), N, jnp.int32) + lanes

    for t in range(NG):
        rec_v[pl.ds(0 * D + t * 16, 16)] = jnp.zeros((16,), jnp.float32)
        rec_v[pl.ds(1 * D + t * 16, 16)] = jnp.zeros((16,), jnp.float32)
        rec_v[pl.ds(2 * D + t * 16, 16)] = jnp.full((16,), -jnp.inf, jnp.float32)
        rec_v[pl.ds(3 * D + t * 16, 16)] = jnp.full((16,), jnp.inf, jnp.float32)

    def edge_body(j, carry, base, m_v, d_v):
        (cur, own, p, ids, rid) = carry[:5]
        accs = carry[5]
        d = d_v[pl.ds(j, 16)][0]
        is_new = d != cur
        real = jnp.logical_and(is_new, cur >= 0)
        flush = jnp.logical_and(real, own)
        head = jnp.logical_and(real, jnp.logical_not(own))

        @pl.when(flush)
        def _stage():
            for t in range(NG):
                st_s[pl.ds(p * D + t * 16, 16)] = accs[0][t]
                st_q[pl.ds(p * D + t * 16, 16)] = accs[1][t]
                st_x[pl.ds(p * D + t * 16, 16)] = accs[2][t]
                st_n[pl.ds(p * D + t * 16, 16)] = accs[3][t]

        @pl.when(head)
        def _head_rec():
            for t in range(NG):
                rec_v[pl.ds(0 * D + t * 16, 16)] = accs[0][t]
                rec_v[pl.ds(1 * D + t * 16, 16)] = accs[1][t]
                rec_v[pl.ds(2 * D + t * 16, 16)] = accs[2][t]
                rec_v[pl.ds(3 * D + t * 16, 16)] = accs[3][t]

        rid2 = jnp.where(head, jnp.where(lanes == 0, cur, rid), rid)
        ids2 = jnp.where(flush, jnp.where(lanes == p, cur, ids), ids)
        p2 = jnp.where(flush, p + 1, p)
        do_drain = jnp.logical_and(flush, p2 == K)

        @pl.when(do_drain)
        def _drain():
            ids_v[pl.ds(0, 16)] = ids2
            hs = []
            for r in range(K):
                idr = ids_v[pl.ds(r, 16)][0]
                rsl = pl.ds(r * D, D)
                hs.append(pltpu.async_copy(st_s.at[rsl], s1_hbm.at[idr], sem1))
                hs.append(pltpu.async_copy(st_q.at[rsl], s2_hbm.at[idr], sem2))
                hs.append(pltpu.async_copy(st_x.at[rsl], mx_hbm.at[idr], sem3))
                hs.append(pltpu.async_copy(st_n.at[rsl], mn_hbm.at[idr], sem4))
            for h in hs:
                h.wait()

        p3 = jnp.where(do_drain, 0, p2)
        ids3 = jnp.where(do_drain, dump, ids2)
        own2 = jnp.logical_or(own, real)
        cur2 = jnp.where(is_new, d, cur)

        new_accs = ([], [], [], [])
        for t in range(NG):
            v = m_v[pl.ds(j * D + t * 16, 16)]
            q = v * v
            new_accs[0].append(jnp.where(is_new, v, accs[0][t] + v))
            new_accs[1].append(jnp.where(is_new, q, accs[1][t] + q))
            new_accs[2].append(jnp.where(is_new, v, jnp.maximum(accs[2][t], v)))
            new_accs[3].append(jnp.where(is_new, v, jnp.minimum(accs[3][t], v)))
        return (cur2, own2, p3, ids3, rid2, new_accs)

    def chunk_pair(k2, carry):
        for b in range(2):
            c = k2 * 2 + b
            base = start + c * CH
            wait_chunk(c, b)

            @pl.when(c + 1 < nch)
            def _pref(c=c, b=b):
                load_chunk(c + 1, 1 - b)

            carry = lax.fori_loop(
                0, CH,
                lambda j, cr, base=base, b=b: edge_body(
                    j, cr, base, m_bufs[b], d_bufs[b]),
                carry)
        return carry

    zero = jnp.zeros((16,), jnp.float32)
    accs0 = ([zero] * NG, [zero] * NG, [zero] * NG, [zero] * NG)
    init = (jnp.int32(-1), jnp.bool_(False), jnp.int32(0), dump, dump, accs0)
    load_chunk(0, 0)
    carry = lax.fori_loop(0, nch // 2, chunk_pair, init)
    (final_cur, final_own, final_p, final_ids, final_rid, final_accs) = carry

    for t in range(NG):
        rec_v[pl.ds(4 * D + t * 16, 16)] = final_accs[0][t]
        rec_v[pl.ds(5 * D + t * 16, 16)] = final_accs[1][t]
        rec_v[pl.ds(6 * D + t * 16, 16)] = final_accs[2][t]
        rec_v[pl.ds(7 * D + t * 16, 16)] = final_accs[3][t]
    rid_f = jnp.where(jnp.logical_not(final_own),
                      jnp.where(lanes == 0, final_cur, final_rid), final_rid)
    rid_f = jnp.where(lanes == 1, final_cur, rid_f)
    rid_v[...] = rid_f
    pltpu.sync_copy(rec_v, rec_hbm.at[pl.ds(wid * 8 * D, 8 * D)])
    pltpu.sync_copy(rid_v, rid_hbm.at[pl.ds(wid * 16, 16)])

    ids_v[pl.ds(0, 16)] = final_ids
    hs = []
    for r in range(K):
        idr = ids_v[pl.ds(r, 16)][0]
        rsl = pl.ds(r * D, D)
        hs.append(pltpu.async_copy(st_s.at[rsl], s1_hbm.at[idr], sem1))
        hs.append(pltpu.async_copy(st_q.at[rsl], s2_hbm.at[idr], sem2))
        hs.append(pltpu.async_copy(st_x.at[rsl], mx_hbm.at[idr], sem3))
        hs.append(pltpu.async_copy(st_n.at[rsl], mn_hbm.at[idr], sem4))
    for h in hs:
        h.wait()


def _stats_sc(m, sdst):
    k = functools.partial(
        pl.kernel,
        mesh=_mesh(),
        out_type=[jax.ShapeDtypeStruct((NPAD, D), jnp.float32)
                  for _ in range(4)]
        + [jax.ShapeDtypeStruct((NW * 8 * D,), jnp.float32),
           jax.ShapeDtypeStruct((NW * 16,), jnp.int32)],
        scratch_types=[
            pltpu.VMEM((CH * D,), jnp.float32),
            pltpu.VMEM((CH * D,), jnp.float32),
            pltpu.VMEM((CH + 16,), jnp.int32),
            pltpu.VMEM((CH + 16,), jnp.int32),
            pltpu.VMEM((K * D,), jnp.float32),
            pltpu.VMEM((K * D,), jnp.float32),
            pltpu.VMEM((K * D,), jnp.float32),
            pltpu.VMEM((K * D,), jnp.float32),
            pltpu.VMEM((32,), jnp.int32),
            pltpu.VMEM((8 * D,), jnp.float32),
            pltpu.VMEM((16,), jnp.int32),
            pltpu.SemaphoreType.DMA,
            pltpu.SemaphoreType.DMA,
            pltpu.SemaphoreType.DMA,
            pltpu.SemaphoreType.DMA,
            pltpu.SemaphoreType.DMA,
            pltpu.SemaphoreType.DMA,
        ],
    )(_stats_body)
    return k(m.reshape(E * D), sdst)


NR = 2 * NW
RPT = NPAD // NW


def _merge_body(rec_hbm, rid_hbm, s1i, s2i, mxi, mni,
                s1o, s2o, mxo, mno,
                rec_v, rid_v, mg_s, mg_q, mg_x, mg_n, mid_v, buf_v):
    wid = _wid()
    lanes = lax.iota(jnp.int32, 16)
    pltpu.sync_copy(rec_hbm, rec_v)
    pltpu.sync_copy(rid_hbm, rid_v.at[pl.ds(0, NR * 8)])

    def rec_body(r, carry):
        (cur, q, bank) = carry[:3]
        accs = carry[3]
        trow = (r // 2) * 16 + (r % 2)
        idr = rid_v[pl.ds(trow, 16)][0]
        is_new = idr != cur
        flushq = jnp.logical_and(is_new, cur >= 0)

        @pl.when(flushq)
        def _fl():
            for t in range(NG):
                mg_s[pl.ds(q * D + t * 16, 16)] = accs[0][t]
                mg_q[pl.ds(q * D + t * 16, 16)] = accs[1][t]
                mg_x[pl.ds(q * D + t * 16, 16)] = accs[2][t]
                mg_n[pl.ds(q * D + t * 16, 16)] = accs[3][t]

        bank2 = jnp.where(flushq, jnp.where(lanes == q % 16, cur, bank), bank)
        q2 = jnp.where(flushq, q + 1, q)
        bfull = jnp.logical_and(flushq, q2 % 16 == 0)

        @pl.when(bfull)
        def _bank():
            mid_v[pl.ds(q2 - 16, 16)] = bank2

        bank3 = jnp.where(bfull, jnp.full((16,), N, jnp.int32), bank2)
        cur2 = jnp.where(is_new, idr, cur)
        row = (r // 2) * 8 + (r % 2) * 4
        new_accs = ([], [], [], [])
        for t in range(NG):
            vs = rec_v[pl.ds(row * D + t * 16, 16)]
            vq = rec_v[pl.ds((row + 1) * D + t * 16, 16)]
            vx = rec_v[pl.ds((row + 2) * D + t * 16, 16)]
            vn = rec_v[pl.ds((row + 3) * D + t * 16, 16)]
            new_accs[0].append(jnp.where(is_new, vs, accs[0][t] + vs))
            new_accs[1].append(jnp.where(is_new, vq, accs[1][t] + vq))
            new_accs[2].append(jnp.where(is_new, vx,
                                         jnp.maximum(accs[2][t], vx)))
            new_accs[3].append(jnp.where(is_new, vn,
                                         jnp.minimum(accs[3][t], vn)))
        return (cur2, q2, bank3, new_accs)

    zero = jnp.zeros((16,), jnp.float32)
    accs0 = ([zero] * NG, [zero] * NG, [zero] * NG, [zero] * NG)
    init = (jnp.int32(-1), jnp.int32(0), jnp.full((16,), N, jnp.int32), accs0)
    (cur_f, q_f, bank_f, accs_f) = lax.fori_loop(0, NR, rec_body, init)

    for t in range(NG):
        mg_s[pl.ds(q_f * D + t * 16, 16)] = accs_f[0][t]
        mg_q[pl.ds(q_f * D + t * 16, 16)] = accs_f[1][t]
        mg_x[pl.ds(q_f * D + t * 16, 16)] = accs_f[2][t]
        mg_n[pl.ds(q_f * D + t * 16, 16)] = accs_f[3][t]
    bank_l = jnp.where(lanes == q_f % 16, cur_f, bank_f)
    mid_v[pl.ds((q_f // 16) * 16, 16)] = bank_l
    q_n = q_f + 1

    lo = wid * RPT
    for (s_in, s_out, mg) in ((s1i, s1o, mg_s), (s2i, s2o, mg_q),
                              (mxi, mxo, mg_x), (mni, mno, mg_n)):
        pltpu.sync_copy(s_in.at[pl.ds(lo * D, RPT * D)], buf_v)

        def ov_body(s, carry, mg=mg):
            mid = mid_v[pl.ds(s, 16)][0]
            hit = jnp.logical_and(
                s < q_n,
                jnp.logical_and(mid >= lo, mid < lo + RPT))

            @pl.when(hit)
            def _ov():
                for t in range(NG):
                    buf_v[pl.ds((mid - lo) * D + t * 16, 16)] = (
                        mg[pl.ds(s * D + t * 16, 16)])

            return carry

        lax.fori_loop(0, NR + 1, ov_body, 0)
        pltpu.sync_copy(buf_v, s_out.at[pl.ds(lo * D, RPT * D)])


def _merge_sc(rec, rid, s1, s2, mx, mn):
    k = functools.partial(
        pl.kernel,
        mesh=_mesh(),
        out_type=[jax.ShapeDtypeStruct((NPAD * D,), jnp.float32)
                  for _ in range(4)],
        scratch_types=[
            pltpu.VMEM((NR * 4 * D,), jnp.float32),
            pltpu.VMEM((NR * 8 + 16,), jnp.int32),
            pltpu.VMEM(((NR + 16) * D,), jnp.float32),
            pltpu.VMEM(((NR + 16) * D,), jnp.float32),
            pltpu.VMEM(((NR + 16) * D,), jnp.float32),
            pltpu.VMEM(((NR + 16) * D,), jnp.float32),
            pltpu.VMEM((NR + 32,), jnp.int32),
            pltpu.VMEM((RPT * D,), jnp.float32),
        ],
    )(_merge_body)
    out = k(rec, rid, s1.reshape(NPAD * D), s2.reshape(NPAD * D),
            mx.reshape(NPAD * D), mn.reshape(NPAD * D))
    return [o.reshape(NPAD, D) for o in out]


def _edge_mlp_body(hs_ref, hd_ref, wa_ref, wb_ref, b_ref, o_ref):
    acc = jnp.dot(hs_ref[...], wa_ref[...], preferred_element_type=jnp.float32)
    acc = acc + jnp.dot(hd_ref[...], wb_ref[...], preferred_element_type=jnp.float32)
    o_ref[...] = jnp.maximum(acc + b_ref[...], 0.0)


def _edge_mlp(hs, hd, w_pre, b_pre):
    return pl.pallas_call(
        _edge_mlp_body,
        grid=(E // EB,),
        in_specs=[
            pl.BlockSpec((EB, D), lambda i: (i, 0)),
            pl.BlockSpec((EB, D), lambda i: (i, 0)),
            pl.BlockSpec((D, D), lambda i: (0, 0)),
            pl.BlockSpec((D, D), lambda i: (0, 0)),
            pl.BlockSpec((1, D), lambda i: (0, 0)),
        ],
        out_specs=pl.BlockSpec((EB, D), lambda i: (i, 0)),
        out_shape=jax.ShapeDtypeStruct((E, D), jnp.float32),
    )(hs, hd, w_pre[:D], w_pre[D:], b_pre.reshape(1, D))


def _embed_body(h_ref, w_ref, b_ref, o_ref):
    o_ref[...] = (
        jnp.dot(h_ref[...], w_ref[...], preferred_element_type=jnp.float32)
        + b_ref[...]
    )


def _embed(h, w_h, b_h):
    return pl.pallas_call(
        _embed_body,
        grid=(N // NB,),
        in_specs=[
            pl.BlockSpec((NB, D), lambda i: (i, 0)),
            pl.BlockSpec((D, D), lambda i: (0, 0)),
            pl.BlockSpec((1, D), lambda i: (0, 0)),
        ],
        out_specs=pl.BlockSpec((NB, D), lambda i: (i, 0)),
        out_shape=jax.ShapeDtypeStruct((N, D), jnp.float32),
    )(h, w_h, b_h.reshape(1, D))


def _post_a_body(x_ref, s1_ref, s2_ref, mx_ref, mn_ref, c0_ref, c1_ref,
                 sn_ref, w_ref, b_ref, y_ref, cs_ref, css_ref):
    cnt = c0_ref[...] + c1_ref[...]
    d = jnp.maximum(cnt, 1.0)
    inv_d = 1.0 / d
    has = cnt > 0.0
    mean = jnp.where(has, s1_ref[...] * inv_d, 0.0)
    var = jnp.where(has,
                    jnp.maximum(s2_ref[...] * inv_d - mean * mean, 0.0), 0.0)
    std = jnp.sqrt(var + 1e-5)
    mx = jnp.where(has, mx_ref[...], 0.0)
    mn = jnp.where(has, mn_ref[...], 0.0)
    logd = jnp.log(d + 1.0)
    amp = logd * (1.0 / AVG_D_LOG)
    att = AVG_D_LOG / logd
    w = w_ref[...]

    acc = jnp.dot(x_ref[...], w[0:D], preferred_element_type=jnp.float32)
    acc_a = jnp.zeros_like(acc)
    acc_t = jnp.zeros_like(acc)
    stats = (mean, mx, mn, std)
    for k in range(4):
        s = stats[k]
        acc = acc + jnp.dot(s, w[D + k * D:D + (k + 1) * D],
                            preferred_element_type=jnp.float32)
        acc_a = acc_a + jnp.dot(s, w[5 * D + k * D:5 * D + (k + 1) * D],
                                preferred_element_type=jnp.float32)
        acc_t = acc_t + jnp.dot(s, w[9 * D + k * D:9 * D + (k + 1) * D],
                                preferred_element_type=jnp.float32)
    y = (acc + amp * acc_a + att * acc_t + b_ref[...]) * sn_ref[...]
    y_ref[...] = y

    @pl.when(pl.program_id(0) == 0)
    def _init():
        cs_ref[...] = jnp.zeros_like(cs_ref)
        css_ref[...] = jnp.zeros_like(css_ref)

    cs_ref[...] += jnp.sum(y, axis=0, keepdims=True)
    css_ref[...] += jnp.sum(y * y, axis=0, keepdims=True)


def _post_b_body(x_ref, y_ref, cs_ref, css_ref, o_ref):
    mu = cs_ref[...] * (1.0 / N)
    vv = css_ref[...] * (1.0 / N) - mu * mu
    yn = (y_ref[...] - mu) * jax.lax.rsqrt(vv + 1e-5)
    o_ref[...] = x_ref[...] + jnp.maximum(yn, 0.0)


def _post(x, s1, s2, mx, mn, c0, c1, snorm_n, w_post, b_post):
    grid = (N // NB,)
    nspec = pl.BlockSpec((NB, D), lambda i: (i, 0))
    one_spec = pl.BlockSpec((NB, 1), lambda i: (i, 0))
    col_spec = pl.BlockSpec((1, D), lambda i: (0, 0))
    y, cs, css = pl.pallas_call(
        _post_a_body,
        grid=grid,
        in_specs=[nspec, nspec, nspec, nspec, nspec, one_spec, one_spec,
                  one_spec, pl.BlockSpec((13 * D, D), lambda i: (0, 0)),
                  col_spec],
        out_specs=[nspec, col_spec, col_spec],
        out_shape=[
            jax.ShapeDtypeStruct((N, D), jnp.float32),
            jax.ShapeDtypeStruct((1, D), jnp.float32),
            jax.ShapeDtypeStruct((1, D), jnp.float32),
        ],
    )(x, s1, s2, mx, mn, c0, c1, snorm_n, w_post, b_post.reshape(1, D))
    return pl.pallas_call(
        _post_b_body,
        grid=grid,
        in_specs=[nspec, nspec, col_spec, col_spec],
        out_specs=nspec,
        out_shape=jax.ShapeDtypeStruct((N, D), jnp.float32),
    )(x, y, cs, css)


def kernel(h, edge_index, e, snorm_n, snorm_e, W_h, b_h, W_pre, b_pre,
           W_post, b_post):
    src = edge_index[0].astype(jnp.int32)
    dst = edge_index[1].astype(jnp.int32)
    perm = jnp.argsort(dst)
    sdst = dst[perm]
    ssrc = src[perm]
    cnt2 = _cnt_sc(sdst)
    c0 = cnt2[0].reshape(N, 1)
    c1 = cnt2[1].reshape(N, 1)
    x = _embed(h, W_h, b_h)
    for l in range(L):
        hs, hd = _gather_sc(x, ssrc, sdst)
        m = _edge_mlp(hs, hd, W_pre[l], b_pre[l])
        s1p, s2p, mxp, mnp, rec, rid = _stats_sc(m, sdst)
        s1, s2, mx, mn = _merge_sc(rec, rid, s1p, s2p, mxp, mnp)
        x = _post(x, s1, s2, mx, mn, c0, c1, snorm_n, W_post[l], b_post[l])
    return x


# speedup vs baseline: 2.8627x; 1.0003x over previous
import functools

import jax
import jax.numpy as jnp
from jax import lax
from jax.experimental import pallas as pl
from jax.experimental.pallas import tpu as pltpu
from jax.experimental.pallas import tpu_sc as plsc

N = 10000
E = 320000
D = 128
L = 4
AVG_D_LOG = 3.4965

NW = 32
PT = E // NW
EB = 2560
NB = 2000
CC = 80
CH = 200
K = 16
NPAD = 10240
NG = 8


def _mesh():
    return plsc.VectorSubcoreMesh(core_axis_name="c", subcore_axis_name="s")


def _wid():
    return lax.axis_index("s") * 2 + lax.axis_index("c")


def _cnt_body(dst_hbm, cnt_hbm, di_v, ones_v, z_v, acc_sh):
    sid = lax.axis_index("s")
    cid = lax.axis_index("c")
    wid = sid * 2 + cid

    def fill(i, carry):
        ones_v[pl.ds(i * 16, 16)] = jnp.full((16,), 1.0, jnp.float32)
        return carry

    lax.fori_loop(0, CC // 16, fill, 0)

    def fillz(i, carry):
        z_v[pl.ds(i * 16, 16)] = jnp.zeros((16,), jnp.float32)
        return carry

    lax.fori_loop(0, 2000 // 16, fillz, 0)

    @pl.when(sid == 0)
    def _zero():
        def zc(i, carry):
            pltpu.sync_copy(z_v, acc_sh.at[pl.ds(i * 2000, 2000)])
            return carry

        lax.fori_loop(0, N // 2000, zc, 0)

    plsc.subcore_barrier()

    start = wid * PT

    def chunk(i, carry):
        pltpu.sync_copy(dst_hbm.at[pl.ds(start + i * CC, CC)], di_v)
        pltpu.sync_copy(ones_v, acc_sh.at[di_v], add=True)
        return carry

    lax.fori_loop(0, PT // CC, chunk, 0)
    plsc.subcore_barrier()

    @pl.when(sid == 0)
    def _export():
        pltpu.sync_copy(acc_sh, cnt_hbm.at[cid])


def _cnt_sc(sdst):
    k = functools.partial(
        pl.kernel,
        mesh=_mesh(),
        out_type=jax.ShapeDtypeStruct((2, N), jnp.float32),
        scratch_types=[
            pltpu.VMEM((CC,), jnp.int32),
            pltpu.VMEM((CC,), jnp.float32),
            pltpu.VMEM((2000,), jnp.float32),
            pltpu.VMEM_SHARED((N,), jnp.float32),
        ],
    )(_cnt_body)
    return k(sdst)


GB = 200


def _gather_body(x_hbm, src_hbm, dst_hbm, hs_hbm, hd_hbm,
                 si0, si1, di0, di1, hs0, hs1, hd0, hd1, sg0, sg1, sw):
    start = _wid() * PT
    nch = PT // GB
    sib = (si0, si1)
    dib = (di0, di1)
    hsb = (hs0, hs1)
    hdb = (hd0, hd1)
    sgb = (sg0, sg1)

    def load_idx(c, b):
        base = start + c * GB
        pltpu.async_copy(src_hbm.at[pl.ds(base, GB)], sib[b], sgb[b])
        pltpu.async_copy(dst_hbm.at[pl.ds(base, GB)], dib[b], sgb[b])

    def wait_idx(c, b):
        base = start + c * GB
        pltpu.make_async_copy(src_hbm.at[pl.ds(base, GB)], sib[b],
                              sgb[b]).wait()
        pltpu.make_async_copy(dst_hbm.at[pl.ds(base, GB)], dib[b],
                              sgb[b]).wait()

    load_idx(0, 0)

    def chunk(c2, carry):
        for b in range(2):
            c = c2 * 2 + b
            base = start + c * GB

            @pl.when(c >= 2)
            def _wprev(b=b, c=c):
                prev = start + (c - 2) * GB
                pltpu.make_async_copy(
                    hsb[b], hs_hbm.at[pl.ds(prev, GB)], sw).wait()
                pltpu.make_async_copy(
                    hdb[b], hd_hbm.at[pl.ds(prev, GB)], sw).wait()

            wait_idx(c, b)
            hs = []
            for (off, ln) in ((0, 80), (80, 80), (160, 40)):
                sl = pl.ds(off, ln)
                hs.append(pltpu.async_copy(
                    x_hbm.at[sib[b].at[sl]], hsb[b].at[sl], sgb[b]))
                hs.append(pltpu.async_copy(
                    x_hbm.at[dib[b].at[sl]], hdb[b].at[sl], sgb[b]))

            @pl.when(c + 1 < nch)
            def _pref(b=b, c=c):
                load_idx(c + 1, 1 - b)

            for h in hs:
                h.wait()
            pltpu.async_copy(hsb[b], hs_hbm.at[pl.ds(base, GB)], sw)
            pltpu.async_copy(hdb[b], hd_hbm.at[pl.ds(base, GB)], sw)
        return carry

    lax.fori_loop(0, nch // 2, chunk, 0)
    for c in (nch - 2, nch - 1):
        b = c % 2
        last = start + c * GB
        pltpu.make_async_copy(hsb[b], hs_hbm.at[pl.ds(last, GB)], sw).wait()
        pltpu.make_async_copy(hdb[b], hd_hbm.at[pl.ds(last, GB)], sw).wait()


def _gather_sc(x, ssrc, sdst):
    k = functools.partial(
        pl.kernel,
        mesh=_mesh(),
        out_type=[jax.ShapeDtypeStruct((E, D), jnp.float32),
                  jax.ShapeDtypeStruct((E, D), jnp.float32)],
        scratch_types=[
            pltpu.VMEM((GB,), jnp.int32),
            pltpu.VMEM((GB,), jnp.int32),
            pltpu.VMEM((GB,), jnp.int32),
            pltpu.VMEM((GB,), jnp.int32),
            pltpu.VMEM((GB, D), jnp.float32),
            pltpu.VMEM((GB, D), jnp.float32),
            pltpu.VMEM((GB, D), jnp.float32),
            pltpu.VMEM((GB, D), jnp.float32),
            pltpu.SemaphoreType.DMA,
            pltpu.SemaphoreType.DMA,
            pltpu.SemaphoreType.DMA,
        ],
    )(_gather_body)
    return k(x, ssrc, sdst)


def _stats_body(m_hbm, dst_hbm, s1_hbm, s2_hbm, mx_hbm, mn_hbm,
                rec_hbm, rid_hbm,
                m_v0, m_v1, d_v0, d_v1, st_s, st_q, st_x, st_n, ids_v,
                rec_v, rid_v, sem1, sem2, sem3, sem4, sl0, sl1):
    wid = _wid()
    start = wid * PT
    nch = PT // CH
    m_bufs = (m_v0, m_v1)
    d_bufs = (d_v0, d_v1)
    lsems = (sl0, sl1)

    def load_chunk(c, b):
        base = start + c * CH
        pltpu.async_copy(m_hbm.at[pl.ds(base * D, CH * D)], m_bufs[b],
                         lsems[b])
        pltpu.async_copy(dst_hbm.at[pl.ds(base, CH)],
                         d_bufs[b].at[pl.ds(0, CH)], lsems[b])

    def wait_chunk(c, b):
        base = start + c * CH
        pltpu.make_async_copy(m_hbm.at[pl.ds(base * D, CH * D)], m_bufs[b],
                              lsems[b]).wait()
        pltpu.make_async_copy(dst_hbm.at[pl.ds(base, CH)],
                              d_bufs[b].at[pl.ds(0, CH)], lsems[b]).wait()
    lanes = lax.iota(jnp.int32, 16)
    dump = jnp.full((16,), N, jnp.int32) + lanes

    for t in range(NG):
        rec_v[pl.ds(0 * D + t * 16, 16)] = jnp.zeros((16,), jnp.float32)
        rec_v[pl.ds(1 * D + t * 16, 16)] = jnp.zeros((16,), jnp.float32)
        rec_v[pl.ds(2 * D + t * 16, 16)] = jnp.full((16,), -jnp.inf, jnp.float32)
        rec_v[pl.ds(3 * D + t * 16, 16)] = jnp.full((16,), jnp.inf, jnp.float32)

    def edge_body(j, carry, base, m_v, d_v):
        (cur, own, p, ids, rid) = carry[:5]
        accs = carry[5]
        d = d_v[pl.ds(j, 16)][0]
        is_new = d != cur
        real = jnp.logical_and(is_new, cur >= 0)
        flush = jnp.logical_and(real, own)
        head = jnp.logical_and(real, jnp.logical_not(own))

        @pl.when(flush)
        def _stage():
            for t in range(NG):
                st_s[pl.ds(p * D + t * 16, 16)] = accs[0][t]
                st_q[pl.ds(p * D + t * 16, 16)] = accs[1][t]
                st_x[pl.ds(p * D + t * 16, 16)] = accs[2][t]
                st_n[pl.ds(p * D + t * 16, 16)] = accs[3][t]

        @pl.when(head)
        def _head_rec():
            for t in range(NG):
                rec_v[pl.ds(0 * D + t * 16, 16)] = accs[0][t]
                rec_v[pl.ds(1 * D + t * 16, 16)] = accs[1][t]
                rec_v[pl.ds(2 * D + t * 16, 16)] = accs[2][t]
                rec_v[pl.ds(3 * D + t * 16, 16)] = accs[3][t]

        rid2 = jnp.where(head, jnp.where(lanes == 0, cur, rid), rid)
        ids2 = jnp.where(flush, jnp.where(lanes == p, cur, ids), ids)
        p2 = jnp.where(flush, p + 1, p)
        do_drain = jnp.logical_and(flush, p2 == K)

        @pl.when(do_drain)
        def _drain():
            ids_v[pl.ds(0, 16)] = ids2
            hs = []
            for r in range(K):
                idr = ids_v[pl.ds(r, 16)][0]
                rsl = pl.ds(r * D, D)
                hs.append(pltpu.async_copy(st_s.at[rsl], s1_hbm.at[idr], sem1))
                hs.append(pltpu.async_copy(st_q.at[rsl], s2_hbm.at[idr], sem2))
                hs.append(pltpu.async_copy(st_x.at[rsl], mx_hbm.at[idr], sem3))
                hs.append(pltpu.async_copy(st_n.at[rsl], mn_hbm.at[idr], sem4))
            for h in hs:
                h.wait()

        p3 = jnp.where(do_drain, 0, p2)
        ids3 = jnp.where(do_drain, dump, ids2)
        own2 = jnp.logical_or(own, real)
        cur2 = jnp.where(is_new, d, cur)

        new_accs = ([], [], [], [])
        for t in range(NG):
            v = m_v[pl.ds(j * D + t * 16, 16)]
            q = v * v
            new_accs[0].append(jnp.where(is_new, v, accs[0][t] + v))
            new_accs[1].append(jnp.where(is_new, q, accs[1][t] + q))
            new_accs[2].append(jnp.where(is_new, v, jnp.maximum(accs[2][t], v)))
            new_accs[3].append(jnp.where(is_new, v, jnp.minimum(accs[3][t], v)))
        return (cur2, own2, p3, ids3, rid2, new_accs)

    def chunk_pair(k2, carry):
        for b in range(2):
            c = k2 * 2 + b
            base = start + c * CH
            wait_chunk(c, b)

            @pl.when(c + 1 < nch)
            def _pref(c=c, b=b):
                load_chunk(c + 1, 1 - b)

            carry = lax.fori_loop(
                0, CH,
                lambda j, cr, base=base, b=b: edge_body(
                    j, cr, base, m_bufs[b], d_bufs[b]),
                carry)
        return carry

    zero = jnp.zeros((16,), jnp.float32)
    accs0 = ([zero] * NG, [zero] * NG, [zero] * NG, [zero] * NG)
    init = (jnp.int32(-1), jnp.bool_(False), jnp.int32(0), dump, dump, accs0)
    load_chunk(0, 0)
    carry = lax.fori_loop(0, nch // 2, chunk_pair, init)
    (final_cur, final_own, final_p, final_ids, final_rid, final_accs) = carry

    for t in range(NG):
        rec_v[pl.ds(4 * D + t * 16, 16)] = final_accs[0][t]
        rec_v[pl.ds(5 * D + t * 16, 16)] = final_accs[1][t]
        rec_v[pl.ds(6 * D + t * 16, 16)] = final_accs[2][t]
        rec_v[pl.ds(7 * D + t * 16, 16)] = final_accs[3][t]
    rid_f = jnp.where(jnp.logical_not(final_own),
                      jnp.where(lanes == 0, final_cur, final_rid), final_rid)
    rid_f = jnp.where(lanes == 1, final_cur, rid_f)
    rid_v[...] = rid_f
    pltpu.sync_copy(rec_v, rec_hbm.at[pl.ds(wid * 8 * D, 8 * D)])
    pltpu.sync_copy(rid_v, rid_hbm.at[pl.ds(wid * 16, 16)])

    ids_v[pl.ds(0, 16)] = final_ids
    hs = []
    for r in range(K):
        idr = ids_v[pl.ds(r, 16)][0]
        rsl = pl.ds(r * D, D)
        hs.append(pltpu.async_copy(st_s.at[rsl], s1_hbm.at[idr], sem1))
        hs.append(pltpu.async_copy(st_q.at[rsl], s2_hbm.at[idr], sem2))
        hs.append(pltpu.async_copy(st_x.at[rsl], mx_hbm.at[idr], sem3))
        hs.append(pltpu.async_copy(st_n.at[rsl], mn_hbm.at[idr], sem4))
    for h in hs:
        h.wait()


def _stats_sc(m, sdst):
    k = functools.partial(
        pl.kernel,
        mesh=_mesh(),
        out_type=[jax.ShapeDtypeStruct((NPAD, D), jnp.float32)
                  for _ in range(4)]
        + [jax.ShapeDtypeStruct((NW * 8 * D,), jnp.float32),
           jax.ShapeDtypeStruct((NW * 16,), jnp.int32)],
        scratch_types=[
            pltpu.VMEM((CH * D,), jnp.float32),
            pltpu.VMEM((CH * D,), jnp.float32),
            pltpu.VMEM((CH + 16,), jnp.int32),
            pltpu.VMEM((CH + 16,), jnp.int32),
            pltpu.VMEM((K * D,), jnp.float32),
            pltpu.VMEM((K * D,), jnp.float32),
            pltpu.VMEM((K * D,), jnp.float32),
            pltpu.VMEM((K * D,), jnp.float32),
            pltpu.VMEM((32,), jnp.int32),
            pltpu.VMEM((8 * D,), jnp.float32),
            pltpu.VMEM((16,), jnp.int32),
            pltpu.SemaphoreType.DMA,
            pltpu.SemaphoreType.DMA,
            pltpu.SemaphoreType.DMA,
            pltpu.SemaphoreType.DMA,
            pltpu.SemaphoreType.DMA,
            pltpu.SemaphoreType.DMA,
        ],
    )(_stats_body)
    return k(m.reshape(E * D), sdst)


NR = 2 * NW
RPT = NPAD // NW


def _merge_body(rec_hbm, rid_hbm, s1i, s2i, mxi, mni,
                s1o, s2o, mxo, mno,
                rec_v, rid_v, mg_s, mg_q, mg_x, mg_n, mid_v, buf_v):
    wid = _wid()
    lanes = lax.iota(jnp.int32, 16)
    pltpu.sync_copy(rec_hbm, rec_v)
    pltpu.sync_copy(rid_hbm, rid_v.at[pl.ds(0, NR * 8)])

    def rec_body(r, carry):
        (cur, q, bank) = carry[:3]
        accs = carry[3]
        trow = (r // 2) * 16 + (r % 2)
        idr = rid_v[pl.ds(trow, 16)][0]
        is_new = idr != cur
        flushq = jnp.logical_and(is_new, cur >= 0)

        @pl.when(flushq)
        def _fl():
            for t in range(NG):
                mg_s[pl.ds(q * D + t * 16, 16)] = accs[0][t]
                mg_q[pl.ds(q * D + t * 16, 16)] = accs[1][t]
                mg_x[pl.ds(q * D + t * 16, 16)] = accs[2][t]
                mg_n[pl.ds(q * D + t * 16, 16)] = accs[3][t]

        bank2 = jnp.where(flushq, jnp.where(lanes == q % 16, cur, bank), bank)
        q2 = jnp.where(flushq, q + 1, q)
        bfull = jnp.logical_and(flushq, q2 % 16 == 0)

        @pl.when(bfull)
        def _bank():
            mid_v[pl.ds(q2 - 16, 16)] = bank2

        bank3 = jnp.where(bfull, jnp.full((16,), N, jnp.int32), bank2)
        cur2 = jnp.where(is_new, idr, cur)
        row = (r // 2) * 8 + (r % 2) * 4
        new_accs = ([], [], [], [])
        for t in range(NG):
            vs = rec_v[pl.ds(row * D + t * 16, 16)]
            vq = rec_v[pl.ds((row + 1) * D + t * 16, 16)]
            vx = rec_v[pl.ds((row + 2) * D + t * 16, 16)]
            vn = rec_v[pl.ds((row + 3) * D + t * 16, 16)]
            new_accs[0].append(jnp.where(is_new, vs, accs[0][t] + vs))
            new_accs[1].append(jnp.where(is_new, vq, accs[1][t] + vq))
            new_accs[2].append(jnp.where(is_new, vx,
                                         jnp.maximum(accs[2][t], vx)))
            new_accs[3].append(jnp.where(is_new, vn,
                                         jnp.minimum(accs[3][t], vn)))
        return (cur2, q2, bank3, new_accs)

    zero = jnp.zeros((16,), jnp.float32)
    accs0 = ([zero] * NG, [zero] * NG, [zero] * NG, [zero] * NG)
    init = (jnp.int32(-1), jnp.int32(0), jnp.full((16,), N, jnp.int32), accs0)
    (cur_f, q_f, bank_f, accs_f) = lax.fori_loop(0, NR, rec_body, init)

    for t in range(NG):
        mg_s[pl.ds(q_f * D + t * 16, 16)] = accs_f[0][t]
        mg_q[pl.ds(q_f * D + t * 16, 16)] = accs_f[1][t]
        mg_x[pl.ds(q_f * D + t * 16, 16)] = accs_f[2][t]
        mg_n[pl.ds(q_f * D + t * 16, 16)] = accs_f[3][t]
    bank_l = jnp.where(lanes == q_f % 16, cur_f, bank_f)
    mid_v[pl.ds((q_f // 16) * 16, 16)] = bank_l
    q_n = q_f + 1

    lo = wid * RPT
    for (s_in, s_out, mg) in ((s1i, s1o, mg_s), (s2i, s2o, mg_q),
                              (mxi, mxo, mg_x), (mni, mno, mg_n)):
        pltpu.sync_copy(s_in.at[pl.ds(lo * D, RPT * D)], buf_v)

        def ov_body(s, carry, mg=mg):
            mid = mid_v[pl.ds(s, 16)][0]
            hit = jnp.logical_and(
                s < q_n,
                jnp.logical_and(mid >= lo, mid < lo + RPT))

            @pl.when(hit)
            def _ov():
                for t in range(NG):
                    buf_v[pl.ds((mid - lo) * D + t * 16, 16)] = (
                        mg[pl.ds(s * D + t * 16, 16)])

            return carry

        lax.fori_loop(0, NR + 1, ov_body, 0)
        pltpu.sync_copy(buf_v, s_out.at[pl.ds(lo * D, RPT * D)])


def _merge_sc(rec, rid, s1, s2, mx, mn):
    k = functools.partial(
        pl.kernel,
        mesh=_mesh(),
        out_type=[jax.ShapeDtypeStruct((NPAD * D,), jnp.float32)
                  for _ in range(4)],
        scratch_types=[
            pltpu.VMEM((NR * 4 * D,), jnp.float32),
            pltpu.VMEM((NR * 8 + 16,), jnp.int32),
            pltpu.VMEM(((NR + 16) * D,), jnp.float32),
            pltpu.VMEM(((NR + 16) * D,), jnp.float32),
            pltpu.VMEM(((NR + 16) * D,), jnp.float32),
            pltpu.VMEM(((NR + 16) * D,), jnp.float32),
            pltpu.VMEM((NR + 32,), jnp.int32),
            pltpu.VMEM((RPT * D,), jnp.float32),
        ],
    )(_merge_body)
    out = k(rec, rid, s1.reshape(NPAD * D), s2.reshape(NPAD * D),
            mx.reshape(NPAD * D), mn.reshape(NPAD * D))
    return [o.reshape(NPAD, D) for o in out]


def _edge_mlp_body(hs_ref, hd_ref, wa_ref, wb_ref, b_ref, o_ref):
    acc = jnp.dot(hs_ref[...], wa_ref[...], preferred_element_type=jnp.float32)
    acc = acc + jnp.dot(hd_ref[...], wb_ref[...], preferred_element_type=jnp.float32)
    o_ref[...] = jnp.maximum(acc + b_ref[...], 0.0)


def _edge_mlp(hs, hd, w_pre, b_pre):
    return pl.pallas_call(
        _edge_mlp_body,
        grid=(E // EB,),
        in_specs=[
            pl.BlockSpec((EB, D), lambda i: (i, 0)),
            pl.BlockSpec((EB, D), lambda i: (i, 0)),
            pl.BlockSpec((D, D), lambda i: (0, 0)),
            pl.BlockSpec((D, D), lambda i: (0, 0)),
            pl.BlockSpec((1, D), lambda i: (0, 0)),
        ],
        out_specs=pl.BlockSpec((EB, D), lambda i: (i, 0)),
        out_shape=jax.ShapeDtypeStruct((E, D), jnp.float32),
    )(hs, hd, w_pre[:D], w_pre[D:], b_pre.reshape(1, D))


def _embed_body(h_ref, w_ref, b_ref, o_ref):
    o_ref[...] = (
        jnp.dot(h_ref[...], w_ref[...], preferred_element_type=jnp.float32)
        + b_ref[...]
    )


def _embed(h, w_h, b_h):
    return pl.pallas_call(
        _embed_body,
        grid=(N // NB,),
        in_specs=[
            pl.BlockSpec((NB, D), lambda i: (i, 0)),
            pl.BlockSpec((D, D), lambda i: (0, 0)),
            pl.BlockSpec((1, D), lambda i: (0, 0)),
        ],
        out_specs=pl.BlockSpec((NB, D), lambda i: (i, 0)),
        out_shape=jax.ShapeDtypeStruct((N, D), jnp.float32),
    )(h, w_h, b_h.reshape(1, D))


def _post_a_body(x_ref, s1_ref, s2_ref, mx_ref, mn_ref, c0_ref, c1_ref,
                 sn_ref, w_ref, b_ref, y_ref, cs_ref, css_ref):
    cnt = c0_ref[...] + c1_ref[...]
    d = jnp.maximum(cnt, 1.0)
    inv_d = 1.0 / d
    has = cnt > 0.0
    mean = jnp.where(has, s1_ref[...] * inv_d, 0.0)
    var = jnp.where(has,
                    jnp.maximum(s2_ref[...] * inv_d - mean * mean, 0.0), 0.0)
    std = jnp.sqrt(var + 1e-5)
    mx = jnp.where(has, mx_ref[...], 0.0)
    mn = jnp.where(has, mn_ref[...], 0.0)
    logd = jnp.log(d + 1.0)
    amp = logd * (1.0 / AVG_D_LOG)
    att = AVG_D_LOG / logd
    w = w_ref[...]

    acc = jnp.dot(x_ref[...], w[0:D], preferred_element_type=jnp.float32)
    acc_a = jnp.zeros_like(acc)
    acc_t = jnp.zeros_like(acc)
    stats = (mean, mx, mn, std)
    for k in range(4):
        s = stats[k]
        acc = acc + jnp.dot(s, w[D + k * D:D + (k + 1) * D],
                            preferred_element_type=jnp.float32)
        acc_a = acc_a + jnp.dot(s, w[5 * D + k * D:5 * D + (k + 1) * D],
                                preferred_element_type=jnp.float32)
        acc_t = acc_t + jnp.dot(s, w[9 * D + k * D:9 * D + (k + 1) * D],
                                preferred_element_type=jnp.float32)
    y = (acc + amp * acc_a + att * acc_t + b_ref[...]) * sn_ref[...]
    y_ref[...] = y

    @pl.when(pl.program_id(0) == 0)
    def _init():
        cs_ref[...] = jnp.zeros_like(cs_ref)
        css_ref[...] = jnp.zeros_like(css_ref)

    cs_ref[...] += jnp.sum(y, axis=0, keepdims=True)
    css_ref[...] += jnp.sum(y * y, axis=0, keepdims=True)


def _post_b_body(x_ref, y_ref, cs_ref, css_ref, o_ref):
    mu = cs_ref[...] * (1.0 / N)
    vv = css_ref[...] * (1.0 / N) - mu * mu
    yn = (y_ref[...] - mu) * jax.lax.rsqrt(vv + 1e-5)
    o_ref[...] = x_ref[...] + jnp.maximum(yn, 0.0)


def _post(x, s1, s2, mx, mn, c0, c1, snorm_n, w_post, b_post):
    grid = (N // NB,)
    nspec = pl.BlockSpec((NB, D), lambda i: (i, 0))
    one_spec = pl.BlockSpec((NB, 1), lambda i: (i, 0))
    col_spec = pl.BlockSpec((1, D), lambda i: (0, 0))
    y, cs, css = pl.pallas_call(
        _post_a_body,
        grid=grid,
        in_specs=[nspec, nspec, nspec, nspec, nspec, one_spec, one_spec,
                  one_spec, pl.BlockSpec((13 * D, D), lambda i: (0, 0)),
                  col_spec],
        out_specs=[nspec, col_spec, col_spec],
        out_shape=[
            jax.ShapeDtypeStruct((N, D), jnp.float32),
            jax.ShapeDtypeStruct((1, D), jnp.float32),
            jax.ShapeDtypeStruct((1, D), jnp.float32),
        ],
    )(x, s1, s2, mx, mn, c0, c1, snorm_n, w_post, b_post.reshape(1, D))
    return pl.pallas_call(
        _post_b_body,
        grid=grid,
        in_specs=[nspec, nspec, col_spec, col_spec],
        out_specs=nspec,
        out_shape=jax.ShapeDtypeStruct((N, D), jnp.float32),
    )(x, y, cs, css)


def kernel(h, edge_index, e, snorm_n, snorm_e, W_h, b_h, W_pre, b_pre,
           W_post, b_post):
    src = edge_index[0].astype(jnp.int32)
    dst = edge_index[1].astype(jnp.int32)
    perm = jnp.argsort(dst)
    sdst = dst[perm]
    ssrc = src[perm]
    cnt2 = _cnt_sc(sdst)
    c0 = cnt2[0].reshape(N, 1)
    c1 = cnt2[1].reshape(N, 1)
    x = _embed(h, W_h, b_h)
    for l in range(L):
        hs, hd = _gather_sc(x, ssrc, sdst)
        m = _edge_mlp(hs, hd, W_pre[l], b_pre[l])
        s1p, s2p, mxp, mnp, rec, rid = _stats_sc(m, sdst)
        s1, s2, mx, mn = _merge_sc(rec, rid, s1p, s2p, mxp, mnp)
        x = _post(x, s1, s2, mx, mn, c0, c1, snorm_n, W_post[l], b_post[l])
    return x
